# baseline (speedup 1.0000x reference)
import functools

import jax
import jax.numpy as jnp
from jax import lax
from jax.experimental import pallas as pl
from jax.experimental.pallas import tpu as pltpu
from jax.experimental.pallas import tpu_sc as plsc

N = 10000
E = 160000
D = 256
H = 256
L = 4
B = 64
FH = 512
OUT = 128
EPS = 1e-5

EP = 163840
NCH = EP // 128
NW = 32
CHW = NCH // NW
CHT = NCH // 16
BE = 1024
NBLK = EP // BE
NROW_OFF = 624
NROW_CP = 640

_MESH = plsc.VectorSubcoreMesh(core_axis_name="c", subcore_axis_name="s")


def _mm(a, b):
    return jax.lax.dot_general(a, b, (((1,), (0,)), ((), ())),
                               preferred_element_type=jnp.float32)


def _mmb(a, b):
    return jax.lax.dot_general(a.astype(jnp.bfloat16), b,
                               (((1,), (0,)), ((), ())),
                               preferred_element_type=jnp.float32)


CH2 = 128
NCH2 = EP // CH2
CHW_A = 56
CHW_B = NCH2 // 16 - CHW_A


def _gather_body(tabA, tabB, src2d, dst2d, gsrc, gdst, idxs, idxd,
                 bufA, bufB, semG):
    c = lax.axis_index("c")
    s = lax.axis_index("s")
    base = jnp.where(c == 0, s * CHW_A, 16 * CHW_A + s * CHW_B)

    @pl.when(c == 0)
    def _():
        pltpu.sync_copy(src2d.at[pl.ds(s * CHW_A, CHW_A)],
                        idxs.at[pl.ds(0, CHW_A)])
        pltpu.sync_copy(dst2d.at[pl.ds(s * CHW_A, CHW_A)],
                        idxd.at[pl.ds(0, CHW_A)])

    @pl.when(c == 1)
    def _():
        pltpu.sync_copy(src2d.at[pl.ds(16 * CHW_A + s * CHW_B, CHW_B)],
                        idxs.at[pl.ds(0, CHW_B)])
        pltpu.sync_copy(dst2d.at[pl.ds(16 * CHW_A + s * CHW_B, CHW_B)],
                        idxd.at[pl.ds(0, CHW_B)])

    def body(i, _):
        c0 = 2 * i
        c1 = 2 * i + 1
        dA0 = pltpu.async_copy(tabA.at[idxs.at[c0]], bufA.at[0], semG)
        dA1 = pltpu.async_copy(tabA.at[idxs.at[c1]], bufA.at[1], semG)
        dB0 = pltpu.async_copy(tabB.at[idxd.at[c0]], bufB.at[0], semG)
        dB1 = pltpu.async_copy(tabB.at[idxd.at[c1]], bufB.at[1], semG)
        dA0.wait()
        dA1.wait()
        dB0.wait()
        dB1.wait()
        row0 = (base + c0) * CH2
        pltpu.sync_copy(bufA.at[0], gsrc.at[pl.ds(row0, CH2)])
        pltpu.sync_copy(bufA.at[1], gsrc.at[pl.ds(row0 + CH2, CH2)])
        pltpu.sync_copy(bufB.at[0], gdst.at[pl.ds(row0, CH2)])
        pltpu.sync_copy(bufB.at[1], gdst.at[pl.ds(row0 + CH2, CH2)])
        return 0

    npairs = jnp.where(c == 0, CHW_A // 2, CHW_B // 2)
    lax.fori_loop(0, npairs, body, 0)


_gather = pl.kernel(
    _gather_body,
    out_type=[jax.ShapeDtypeStruct((EP, D), jnp.uint32),
              jax.ShapeDtypeStruct((EP, H // 2), jnp.uint32)],
    mesh=_MESH,
    scratch_types=[pltpu.VMEM((CHW_A, CH2), jnp.int32),
                   pltpu.VMEM((CHW_A, CH2), jnp.int32),
                   pltpu.VMEM((2, CH2, D), jnp.uint32),
                   pltpu.VMEM((2, CH2, H // 2), jnp.uint32),
                   pltpu.SemaphoreType.DMA],
)


def _scatter_body(msg, dst2d, zrows, nagg, idxd, buf, accum, semR):
    c = lax.axis_index("c")
    s = lax.axis_index("s")
    coff = c * 128
    pltpu.sync_copy(zrows, accum.at[pl.ds(s * NROW_OFF, NROW_CP)])
    plsc.subcore_barrier()
    pltpu.sync_copy(dst2d.at[pl.ds(s * CHT, CHT)], idxd)

    def body(i, _):
        c0 = 2 * i
        c1 = 2 * i + 1
        row0 = s * (CHT * 128) + c0 * 128
        d0 = pltpu.async_copy(msg.at[pl.ds(row0, 128), pl.ds(coff, 128)],
                              buf.at[0], semR)
        d1 = pltpu.async_copy(msg.at[pl.ds(row0 + 128, 128),
                                     pl.ds(coff, 128)], buf.at[1], semR)
        d0.wait()
        d1.wait()
        pltpu.sync_copy(buf.at[0], accum.at[idxd.at[c0]], add=True)
        pltpu.sync_copy(buf.at[1], accum.at[idxd.at[c1]], add=True)
        return 0

    lax.fori_loop(0, CHT // 2, body, 0)
    plsc.subcore_barrier()
    pltpu.sync_copy(accum.at[pl.ds(s * NROW_OFF, NROW_CP)],
                    nagg.at[pl.ds(s * NROW_OFF, NROW_CP), pl.ds(coff, 128)])


_scatter = pl.kernel(
    _scatter_body,
    out_type=jax.ShapeDtypeStruct((N, D), jnp.float32),
    mesh=_MESH,
    scratch_types=[pltpu.VMEM((CHT, 128), jnp.int32),
                   pltpu.VMEM((2, 128, 128), jnp.float32),
                   pltpu.MemorySpace.VMEM_SHARED((N, 128), jnp.float32),
                   pltpu.SemaphoreType.DMA],
)


def _rb(x):
    return jax.lax.bitcast_convert_type(
        x.astype(jnp.bfloat16).astype(jnp.float32), jnp.uint32)


def _pack2(hi, lo):
    return jax.lax.bitwise_or(
        _rb(hi), jax.lax.shift_right_logical(_rb(lo), jnp.uint32(16)))


def _lo_f32(pk):
    return jax.lax.bitcast_convert_type(
        jax.lax.shift_left(pk, jnp.uint32(16)), jnp.float32)


def _hi_f32(pk):
    return jax.lax.bitcast_convert_type(
        jax.lax.bitwise_and(pk, jnp.uint32(0xFFFF0000)), jnp.float32)


def _node_pre_body(x_ref, w_ref, b_ref, tabA_ref, tabB_ref):
    t = _mmb(x_ref[...], w_ref[...]) + b_ref[0:1, :]
    xm = t[:, :D]
    xe = t[:, D:]
    tabA_ref[...] = _pack2(xe, xm)
    tabB_ref[...] = _pack2(xe[:, H // 2:], xe[:, :H // 2])


def _node_pre(x, wcat, bcat):
    return pl.pallas_call(
        _node_pre_body,
        grid=(5,),
        in_specs=[pl.BlockSpec((2000, D), lambda i: (i, 0)),
                  pl.BlockSpec((D, 2 * D), lambda i: (0, 0)),
                  pl.BlockSpec((8, 2 * D), lambda i: (0, 0))],
        out_specs=[pl.BlockSpec((2000, D), lambda i: (i, 0)),
                   pl.BlockSpec((2000, H // 2), lambda i: (i, 0))],
        out_shape=[jax.ShapeDtypeStruct((N, D), jnp.uint32),
                   jax.ShapeDtypeStruct((N, H // 2), jnp.uint32)],
    )(x, wcat, bcat)


def _bn_from_stats(t, stats_ref, g, be):
    m = stats_ref[0:1, :] / E
    v = stats_ref[1:2, :] / E - m * m
    inv = jax.lax.rsqrt(v + EPS)
    return jnp.maximum(g * (t - m) * inv + be, 0.0)


def _unpack2(pk):
    return jnp.concatenate([_lo_f32(pk), _hi_f32(pk)], axis=1)


def _pack_cols(t):
    w = t.shape[1] // 2
    return _pack2(t[:, w:], t[:, :w])


def _edge1_body(apply_bn, epre_ref, gsrc_ref, gdst_ref, wcat_ref, vec_ref,
                pstats_ref, msg_ref, eh1_ref, stats_ref, acc):
    i = pl.program_id(0)

    @pl.when(i == 0)
    def _():
        acc[...] = jnp.zeros((8, H), jnp.float32)

    if apply_bn:
        ep = _unpack2(epre_ref[...])
        ebn = _bn_from_stats(ep, pstats_ref, vec_ref[1:2, :], vec_ref[2:3, :])
    else:
        ebn = epre_ref[...]
    t = _mmb(ebn, wcat_ref[...])
    gp = gsrc_ref[...]
    dp = gdst_ref[...]
    xm = _lo_f32(gp)
    xs = _hi_f32(gp)
    xd = jnp.concatenate([_lo_f32(dp), _hi_f32(dp)], axis=1)
    rows = i * BE + jax.lax.broadcasted_iota(jnp.int32, (BE, 1), 0)
    mask = rows < E
    msg = jnp.maximum(t[:, :D] + xm, 0.0)
    msg_ref[...] = jnp.where(mask, msg, 0.0)
    eh1 = t[:, D:] + xs + xd + vec_ref[0:1, :]
    eh1_ref[...] = _pack_cols(eh1)
    mm_ = jnp.where(mask, eh1, 0.0)
    acc[0:1, :] = acc[0:1, :] + jnp.sum(mm_, axis=0, keepdims=True)
    acc[1:2, :] = acc[1:2, :] + jnp.sum(mm_ * mm_, axis=0, keepdims=True)

    @pl.when(i == NBLK - 1)
    def _():
        stats_ref[...] = acc[...]


def _edge1(apply_bn, epre, gsrc, gdst, wcat, vec, pstats):
    ep_w = H // 2 if apply_bn else D
    return pl.pallas_call(
        functools.partial(_edge1_body, apply_bn),
        grid=(NBLK,),
        in_specs=[pl.BlockSpec((BE, ep_w), lambda i: (i, 0)),
                  pl.BlockSpec((BE, D), lambda i: (i, 0)),
                  pl.BlockSpec((BE, H // 2), lambda i: (i, 0)),
                  pl.BlockSpec((D, 2 * D), lambda i: (0, 0)),
                  pl.BlockSpec((8, H), lambda i: (0, 0)),
                  pl.BlockSpec((8, H), lambda i: (0, 0))],
        out_specs=[pl.BlockSpec((BE, D), lambda i: (i, 0)),
                   pl.BlockSpec((BE, H // 2), lambda i: (i, 0)),
                   pl.BlockSpec((8, H), lambda i: (0, 0))],
        out_shape=[jax.ShapeDtypeStruct((EP, D), jnp.float32),
                   jax.ShapeDtypeStruct((EP, H // 2), jnp.uint32),
                   jax.ShapeDtypeStruct((8, H), jnp.float32)],
        scratch_shapes=[pltpu.VMEM((8, H), jnp.float32)],
    )(epre, gsrc, gdst, wcat, vec, pstats)


def _edge2_body(eh1_ref, stats1_ref, vec_ref, w2_ref, eh2_ref, stats2_ref, acc):
    i = pl.program_id(0)

    @pl.when(i == 0)
    def _():
        acc[...] = jnp.zeros((8, H), jnp.float32)

    a = _bn_from_stats(_unpack2(eh1_ref[...]), stats1_ref,
                       vec_ref[0:1, :], vec_ref[1:2, :])
    t = _mmb(a, w2_ref[...]) + vec_ref[2:3, :]
    eh2_ref[...] = _pack_cols(t)
    rows = i * BE + jax.lax.broadcasted_iota(jnp.int32, (BE, 1), 0)
    mask = rows < E
    mm_ = jnp.where(mask, t, 0.0)
    acc[0:1, :] = acc[0:1, :] + jnp.sum(mm_, axis=0, keepdims=True)
    acc[1:2, :] = acc[1:2, :] + jnp.sum(mm_ * mm_, axis=0, keepdims=True)

    @pl.when(i == NBLK - 1)
    def _():
        stats2_ref[...] = acc[...]


def _edge2(eh1, stats1, vec, w2):
    return pl.pallas_call(
        _edge2_body,
        grid=(NBLK,),
        in_specs=[pl.BlockSpec((BE, H // 2), lambda i: (i, 0)),
                  pl.BlockSpec((8, H), lambda i: (0, 0)),
                  pl.BlockSpec((8, H), lambda i: (0, 0)),
                  pl.BlockSpec((H, H), lambda i: (0, 0))],
        out_specs=[pl.BlockSpec((BE, H // 2), lambda i: (i, 0)),
                   pl.BlockSpec((8, H), lambda i: (0, 0))],
        out_shape=[jax.ShapeDtypeStruct((EP, H // 2), jnp.uint32),
                   jax.ShapeDtypeStruct((8, H), jnp.float32)],
        scratch_shapes=[pltpu.VMEM((8, H), jnp.float32)],
    )(eh1, stats1, vec, w2)


def _node_xnew(nagg_ref, x_ref, w1_ref, w2_ref, vec_ref):
    h0 = nagg_ref[...] + x_ref[...]
    y = _mmb(h0, w1_ref[...]) + vec_ref[0:1, :]
    m = jnp.mean(y, axis=0, keepdims=True)
    v = jnp.mean(y * y, axis=0, keepdims=True) - m * m
    h = jnp.maximum(vec_ref[1:2, :] * (y - m) * jax.lax.rsqrt(v + EPS)
                    + vec_ref[2:3, :], 0.0)
    y2 = _mmb(h, w2_ref[...]) + vec_ref[3:4, :]
    m2 = jnp.mean(y2, axis=0, keepdims=True)
    v2 = jnp.mean(y2 * y2, axis=0, keepdims=True) - m2 * m2
    return jnp.maximum(vec_ref[4:5, :] * (y2 - m2)
                       * jax.lax.rsqrt(v2 + EPS) + vec_ref[5:6, :], 0.0)


def _node_body(nagg_ref, x_ref, w1_ref, w2_ref, vec_ref, out_ref):
    out_ref[...] = _node_xnew(nagg_ref, x_ref, w1_ref, w2_ref, vec_ref)


def _node(nagg, x, w1, w2, vec):
    return pl.pallas_call(
        _node_body,
        in_specs=[pl.BlockSpec((N, D), lambda: (0, 0)),
                  pl.BlockSpec((N, D), lambda: (0, 0)),
                  pl.BlockSpec((D, H), lambda: (0, 0)),
                  pl.BlockSpec((H, H), lambda: (0, 0)),
                  pl.BlockSpec((8, H), lambda: (0, 0))],
        out_specs=pl.BlockSpec((N, D), lambda: (0, 0)),
        out_shape=jax.ShapeDtypeStruct((N, D), jnp.float32),
    )(nagg, x, w1, w2, vec)


def _node_fused_body(nagg_ref, x_ref, w1_ref, w2_ref, vec_ref, wcat_ref,
                     bcat_ref, out_ref, tabA_ref, tabB_ref):
    xn = _node_xnew(nagg_ref, x_ref, w1_ref, w2_ref, vec_ref)
    out_ref[...] = xn
    t = _mm(xn, wcat_ref[...]) + bcat_ref[0:1, :]
    tabA_ref[...] = t
    tabB_ref[...] = t[:, D:]


def _node_fused(nagg, x, w1, w2, vec, wcat, bcat):
    return pl.pallas_call(
        _node_fused_body,
        in_specs=[pl.BlockSpec((N, D), lambda: (0, 0)),
                  pl.BlockSpec((N, D), lambda: (0, 0)),
                  pl.BlockSpec((D, H), lambda: (0, 0)),
                  pl.BlockSpec((H, H), lambda: (0, 0)),
                  pl.BlockSpec((8, H), lambda: (0, 0)),
                  pl.BlockSpec((D, 2 * D), lambda: (0, 0)),
                  pl.BlockSpec((8, 2 * D), lambda: (0, 0))],
        out_specs=[pl.BlockSpec((N, D), lambda: (0, 0)),
                   pl.BlockSpec((N, 2 * D), lambda: (0, 0)),
                   pl.BlockSpec((N, H), lambda: (0, 0))],
        out_shape=[jax.ShapeDtypeStruct((N, D), jnp.float32),
                   jax.ShapeDtypeStruct((N, 2 * D), jnp.float32),
                   jax.ShapeDtypeStruct((N, H), jnp.float32)],
    )(nagg, x, w1, w2, vec, wcat, bcat)


def _bnrelu_body(eh2_ref, stats_ref, vec_ref, out_ref):
    i = pl.program_id(0)
    rows = i * BE + jax.lax.broadcasted_iota(jnp.int32, (BE, 1), 0)
    v = _bn_from_stats(_unpack2(eh2_ref[...]), stats_ref,
                       vec_ref[0:1, :], vec_ref[1:2, :])
    out_ref[...] = jnp.where(rows < E, v, 0.0)


def _bnrelu(eh2, stats, vec):
    return pl.pallas_call(
        _bnrelu_body,
        grid=(NBLK,),
        in_specs=[pl.BlockSpec((BE, H // 2), lambda i: (i, 0)),
                  pl.BlockSpec((8, H), lambda i: (0, 0)),
                  pl.BlockSpec((8, H), lambda i: (0, 0))],
        out_specs=pl.BlockSpec((BE, H), lambda i: (i, 0)),
        out_shape=jax.ShapeDtypeStruct((EP, H), jnp.float32),
    )(eh2, stats, vec)


def _pool_body(nblk, limit, width, data_ref, gidx_ref, sum_ref, cnt_ref,
               accs, accc):
    i = pl.program_id(0)

    @pl.when(i == 0)
    def _():
        accs[...] = jnp.zeros((B, width), jnp.float32)
        accc[...] = jnp.zeros((B, 128), jnp.float32)

    d = data_ref[...]
    giota = jax.lax.broadcasted_iota(jnp.int32, (B, 1), 0)
    lane = jax.lax.broadcasted_iota(jnp.int32, (1, 128), 1)
    for r in range(8):
        gr = gidx_ref[r:r + 1, :]
        ids = i * BE + r * 128 + lane
        oh = jnp.where((gr == giota) & (ids < limit), 1.0, 0.0)
        accs[...] = accs[...] + _mm(oh, d[r * 128:(r + 1) * 128, :])
        accc[...] = accc[...] + oh

    @pl.when(i == nblk - 1)
    def _():
        sum_ref[...] = accs[...]
        cnt = jnp.sum(accc[...], axis=1, keepdims=True)
        cnt_ref[...] = jnp.broadcast_to(cnt, (B, 128))


def _pool(limit, data, gidx):
    nblk = data.shape[0] // BE
    width = data.shape[1]
    return pl.pallas_call(
        functools.partial(_pool_body, nblk, limit, width),
        grid=(nblk,),
        in_specs=[pl.BlockSpec((BE, width), lambda i: (i, 0)),
                  pl.BlockSpec((8, 128), lambda i: (i, 0))],
        out_specs=[pl.BlockSpec((B, width), lambda i: (0, 0)),
                   pl.BlockSpec((B, 128), lambda i: (0, 0))],
        out_shape=[jax.ShapeDtypeStruct((B, width), jnp.float32),
                   jax.ShapeDtypeStruct((B, 128), jnp.float32)],
        scratch_shapes=[pltpu.VMEM((B, width), jnp.float32),
                        pltpu.VMEM((B, 128), jnp.float32)],
    )(data, gidx)


def _final_body(pn_ref, cn_ref, pe_ref, ce_ref, w1n_ref, w1e_ref, w2_ref,
                vb_ref, b2_ref, out_ref):
    pn = pn_ref[...] / jnp.maximum(cn_ref[:, 0:1], 1.0)
    pe = pe_ref[...] / jnp.maximum(ce_ref[:, 0:1], 1.0)
    hn = jnp.maximum(_mm(pn, w1n_ref[...]) + vb_ref[0:1, :], 0.0)
    he = jnp.maximum(_mm(pe, w1e_ref[...]) + vb_ref[1:2, :], 0.0)
    out_ref[...] = (_mm(hn, w2_ref[0:FH, :]) + _mm(he, w2_ref[FH:, :])
                    + b2_ref[0:1, :])


def _final(pn, cn, pe, ce, w1n, w1e, w2, vb, b2):
    return pl.pallas_call(
        _final_body,
        in_specs=[pl.BlockSpec((B, H), lambda: (0, 0)),
                  pl.BlockSpec((B, 128), lambda: (0, 0)),
                  pl.BlockSpec((B, H), lambda: (0, 0)),
                  pl.BlockSpec((B, H), lambda: (0, 0)),
                  pl.BlockSpec((H, FH), lambda: (0, 0)),
                  pl.BlockSpec((H, FH), lambda: (0, 0)),
                  pl.BlockSpec((2 * FH, OUT), lambda: (0, 0)),
                  pl.BlockSpec((8, FH), lambda: (0, 0)),
                  pl.BlockSpec((8, OUT), lambda: (0, 0))],
        out_specs=pl.BlockSpec((B, OUT), lambda: (0, 0)),
        out_shape=jax.ShapeDtypeStruct((B, OUT), jnp.float32),
    )(pn, cn, pe, ce, w1n, w1e, w2, vb, b2)


def _vecs8(*rows, width=H):
    out = jnp.zeros((8, width), jnp.float32)
    for r, v in enumerate(rows):
        out = out.at[r, :].set(v)
    return out


def kernel(x, edge_attr, params, edge_index, batch, num_graphs):
    p = params
    src = edge_index[0]
    dst = edge_index[1]
    src_pad = jnp.pad(src, (0, EP - E))
    dst_pad = jnp.pad(dst, (0, EP - E))
    src2d_g = src_pad.reshape(NCH2, CH2)
    dst2d_g = dst_pad.reshape(NCH2, CH2)
    src2d_s = src_pad.reshape(NCH, 128)
    dst2d_s = dst_pad.reshape(NCH, 128)
    epre = jnp.pad(edge_attr, ((0, EP - E), (0, 0)))
    zrows = jnp.zeros((NROW_CP, 128), jnp.float32)
    zstats = jnp.zeros((8, H), jnp.float32)

    def _wcats(l):
        Wx = p["msg_W"][l][:D]
        xcat_W = jnp.concatenate([Wx, p["e1_W"][l]],
                                 axis=1).astype(jnp.bfloat16)
        bcat = _vecs8(jnp.concatenate([p["msg_b"][l], jnp.zeros((H,))]),
                      width=2 * D)
        return xcat_W, bcat

    xcat_W, bcat = _wcats(0)
    tabA, tabB = _node_pre(x, xcat_W, bcat)

    bn_state = None
    for l in range(L):
        We = p["msg_W"][l][D:]
        gsrc, gdst = _gather(tabA, tabB, src2d_g, dst2d_g)

        wcat = jnp.concatenate([We, p["e1_W"][l]],
                               axis=1).astype(jnp.bfloat16)
        if bn_state is None:
            vec1 = _vecs8(p["e1_b"][l])
            msg, eh1, stats1 = _edge1(False, epre, gsrc, gdst, wcat, vec1,
                                      zstats)
        else:
            pstats, pg, pbe = bn_state
            vec1 = _vecs8(p["e1_b"][l], pg, pbe)
            msg, eh1, stats1 = _edge1(True, epre, gsrc, gdst, wcat, vec1,
                                      pstats)

        nagg = _scatter(msg, dst2d_s, zrows)

        vec2 = _vecs8(p["e1_g"][l], p["e1_be"][l], p["e2_b"][l])
        eh2, stats2 = _edge2(eh1, stats1, vec2,
                             p["e2_W"][l].astype(jnp.bfloat16))

        vecn = _vecs8(p["n1_b"][l], p["n1_g"][l], p["n1_be"][l],
                      p["n2_b"][l], p["n2_g"][l], p["n2_be"][l])
        x = _node(nagg, x, p["n1_W"][l].astype(jnp.bfloat16),
                  p["n2_W"][l].astype(jnp.bfloat16), vecn)
        if l < L - 1:
            xcat_W, bcat = _wcats(l + 1)
            tabA, tabB = _node_pre(x, xcat_W, bcat)

        epre = eh2
        bn_state = (stats2, p["e2_g"][l], p["e2_be"][l])

    x_pad = jnp.pad(x, ((0, 10240 - N), (0, 0)))
    batch2d = jnp.pad(batch, (0, 10240 - N), constant_values=B).reshape(80, 128)
    pn_sum, cn = _pool(N, x_pad, batch2d)

    stats2, pg, pbe = bn_state
    efin = _bnrelu(epre, stats2, _vecs8(pg, pbe))
    sn = _scatter(efin, src2d_s, zrows)
    emask = jnp.where(jnp.arange(EP)[:, None] < E, 1.0,
                      0.0).astype(jnp.float32)
    ones_m = jnp.broadcast_to(emask, (EP, D))
    deg = _scatter(ones_m, src2d_s, zrows)
    pe_sum, _ = _pool(N, jnp.pad(sn, ((0, 240, ), (0, 0))), batch2d)
    ce, _ = _pool(N, jnp.pad(deg, ((0, 240), (0, 0))), batch2d)

    vb = _vecs8(p["lin1n_b"], p["lin1e_b"], width=FH)
    b2 = _vecs8(p["lin2_b"], width=OUT)
    out = _final(pn_sum, cn, pe_sum, ce, p["lin1n_W"], p["lin1e_W"],
                 p["lin2_W"], vb, b2)
    return out


# speedup vs baseline: 1.0392x; 1.0392x over previous
import functools

import jax
import jax.numpy as jnp
from jax import lax
from jax.experimental import pallas as pl
from jax.experimental.pallas import tpu as pltpu
from jax.experimental.pallas import tpu_sc as plsc

N = 10000
E = 160000
D = 256
H = 256
L = 4
B = 64
FH = 512
OUT = 128
EPS = 1e-5

EP = 163840
NCH = EP // 128
NW = 32
CHW = NCH // NW
CHT = NCH // 16
BE = 1024
NBLK = EP // BE
NROW_OFF = 624
NROW_CP = 640

_MESH = plsc.VectorSubcoreMesh(core_axis_name="c", subcore_axis_name="s")


def _mm(a, b):
    return jax.lax.dot_general(a, b, (((1,), (0,)), ((), ())),
                               preferred_element_type=jnp.float32)


def _mmb(a, b):
    return jax.lax.dot_general(a.astype(jnp.bfloat16), b,
                               (((1,), (0,)), ((), ())),
                               preferred_element_type=jnp.float32)


CH2 = 128
NCH2 = EP // CH2
CHW_A = 72
CHW_B = NCH2 // 16 - CHW_A


def _gather_body(tabA, tabB, src2d, dst2d, gsrc, gdst, idxs, idxd,
                 bufA, bufB, semG):
    c = lax.axis_index("c")
    s = lax.axis_index("s")
    base = jnp.where(c == 0, s * CHW_A, 16 * CHW_A + s * CHW_B)

    @pl.when(c == 0)
    def _():
        pltpu.sync_copy(src2d.at[pl.ds(s * CHW_A, CHW_A)],
                        idxs.at[pl.ds(0, CHW_A)])
        pltpu.sync_copy(dst2d.at[pl.ds(s * CHW_A, CHW_A)],
                        idxd.at[pl.ds(0, CHW_A)])

    @pl.when(c == 1)
    def _():
        pltpu.sync_copy(src2d.at[pl.ds(16 * CHW_A + s * CHW_B, CHW_B)],
                        idxs.at[pl.ds(0, CHW_B)])
        pltpu.sync_copy(dst2d.at[pl.ds(16 * CHW_A + s * CHW_B, CHW_B)],
                        idxd.at[pl.ds(0, CHW_B)])

    def body(i, _):
        c0 = 2 * i
        c1 = 2 * i + 1
        dA0 = pltpu.async_copy(tabA.at[idxs.at[c0]], bufA.at[0], semG)
        dA1 = pltpu.async_copy(tabA.at[idxs.at[c1]], bufA.at[1], semG)
        dB0 = pltpu.async_copy(tabB.at[idxd.at[c0]], bufB.at[0], semG)
        dB1 = pltpu.async_copy(tabB.at[idxd.at[c1]], bufB.at[1], semG)
        dA0.wait()
        dA1.wait()
        dB0.wait()
        dB1.wait()
        row0 = (base + c0) * CH2
        pltpu.sync_copy(bufA.at[0], gsrc.at[pl.ds(row0, CH2)])
        pltpu.sync_copy(bufA.at[1], gsrc.at[pl.ds(row0 + CH2, CH2)])
        pltpu.sync_copy(bufB.at[0], gdst.at[pl.ds(row0, CH2)])
        pltpu.sync_copy(bufB.at[1], gdst.at[pl.ds(row0 + CH2, CH2)])
        return 0

    npairs = jnp.where(c == 0, CHW_A // 2, CHW_B // 2)
    lax.fori_loop(0, npairs, body, 0)


_gather = pl.kernel(
    _gather_body,
    out_type=[jax.ShapeDtypeStruct((EP, D), jnp.uint32),
              jax.ShapeDtypeStruct((EP, H // 2), jnp.uint32)],
    mesh=_MESH,
    scratch_types=[pltpu.VMEM((CHW_A, CH2), jnp.int32),
                   pltpu.VMEM((CHW_A, CH2), jnp.int32),
                   pltpu.VMEM((2, CH2, D), jnp.uint32),
                   pltpu.VMEM((2, CH2, H // 2), jnp.uint32),
                   pltpu.SemaphoreType.DMA],
)


def _scatter_body(msg, dst2d, zrows, nagg, idxd, buf, accum, semR):
    c = lax.axis_index("c")
    s = lax.axis_index("s")
    coff = c * 128
    pltpu.sync_copy(zrows, accum.at[pl.ds(s * NROW_OFF, NROW_CP)])
    plsc.subcore_barrier()
    pltpu.sync_copy(dst2d.at[pl.ds(s * CHT, CHT)], idxd)

    def body(i, _):
        c0 = 2 * i
        c1 = 2 * i + 1
        row0 = s * (CHT * 128) + c0 * 128
        d0 = pltpu.async_copy(msg.at[pl.ds(row0, 128), pl.ds(coff, 128)],
                              buf.at[0], semR)
        d1 = pltpu.async_copy(msg.at[pl.ds(row0 + 128, 128),
                                     pl.ds(coff, 128)], buf.at[1], semR)
        d0.wait()
        d1.wait()
        pltpu.sync_copy(buf.at[0], accum.at[idxd.at[c0]], add=True)
        pltpu.sync_copy(buf.at[1], accum.at[idxd.at[c1]], add=True)
        return 0

    lax.fori_loop(0, CHT // 2, body, 0)
    plsc.subcore_barrier()
    pltpu.sync_copy(accum.at[pl.ds(s * NROW_OFF, NROW_CP)],
                    nagg.at[pl.ds(s * NROW_OFF, NROW_CP), pl.ds(coff, 128)])


_scatter = pl.kernel(
    _scatter_body,
    out_type=jax.ShapeDtypeStruct((N, D), jnp.float32),
    mesh=_MESH,
    scratch_types=[pltpu.VMEM((CHT, 128), jnp.int32),
                   pltpu.VMEM((2, 128, 128), jnp.float32),
                   pltpu.MemorySpace.VMEM_SHARED((N, 128), jnp.float32),
                   pltpu.SemaphoreType.DMA],
)


def _rb(x):
    return jax.lax.bitcast_convert_type(
        x.astype(jnp.bfloat16).astype(jnp.float32), jnp.uint32)


def _pack2(hi, lo):
    return jax.lax.bitwise_or(
        _rb(hi), jax.lax.shift_right_logical(_rb(lo), jnp.uint32(16)))


def _lo_f32(pk):
    return jax.lax.bitcast_convert_type(
        jax.lax.shift_left(pk, jnp.uint32(16)), jnp.float32)


def _hi_f32(pk):
    return jax.lax.bitcast_convert_type(
        jax.lax.bitwise_and(pk, jnp.uint32(0xFFFF0000)), jnp.float32)


def _node_pre_body(x_ref, w_ref, b_ref, tabA_ref, tabB_ref):
    t = _mmb(x_ref[...], w_ref[...]) + b_ref[0:1, :]
    xm = t[:, :D]
    xe = t[:, D:]
    tabA_ref[...] = _pack2(xe, xm)
    tabB_ref[...] = _pack2(xe[:, H // 2:], xe[:, :H // 2])


def _node_pre(x, wcat, bcat):
    return pl.pallas_call(
        _node_pre_body,
        grid=(5,),
        in_specs=[pl.BlockSpec((2000, D), lambda i: (i, 0)),
                  pl.BlockSpec((D, 2 * D), lambda i: (0, 0)),
                  pl.BlockSpec((8, 2 * D), lambda i: (0, 0))],
        out_specs=[pl.BlockSpec((2000, D), lambda i: (i, 0)),
                   pl.BlockSpec((2000, H // 2), lambda i: (i, 0))],
        out_shape=[jax.ShapeDtypeStruct((N, D), jnp.uint32),
                   jax.ShapeDtypeStruct((N, H // 2), jnp.uint32)],
    )(x, wcat, bcat)


def _bn_from_stats(t, stats_ref, g, be):
    m = stats_ref[0:1, :] / E
    v = stats_ref[1:2, :] / E - m * m
    inv = jax.lax.rsqrt(v + EPS)
    return jnp.maximum(g * (t - m) * inv + be, 0.0)


def _unpack2(pk):
    return jnp.concatenate([_lo_f32(pk), _hi_f32(pk)], axis=1)


def _pack_cols(t):
    w = t.shape[1] // 2
    return _pack2(t[:, w:], t[:, :w])


def _edge1_body(apply_bn, epre_ref, gsrc_ref, gdst_ref, wcat_ref, vec_ref,
                pstats_ref, msg_ref, eh1_ref, stats_ref, acc):
    i = pl.program_id(0)

    @pl.when(i == 0)
    def _():
        acc[...] = jnp.zeros((8, H), jnp.float32)

    if apply_bn:
        ep = _unpack2(epre_ref[...])
        ebn = _bn_from_stats(ep, pstats_ref, vec_ref[1:2, :], vec_ref[2:3, :])
    else:
        ebn = epre_ref[...]
    t = _mmb(ebn, wcat_ref[...])
    gp = gsrc_ref[...]
    dp = gdst_ref[...]
    xm = _lo_f32(gp)
    xs = _hi_f32(gp)
    xd = jnp.concatenate([_lo_f32(dp), _hi_f32(dp)], axis=1)
    rows = i * BE + jax.lax.broadcasted_iota(jnp.int32, (BE, 1), 0)
    mask = rows < E
    msg = jnp.maximum(t[:, :D] + xm, 0.0)
    msg_ref[...] = jnp.where(mask, msg, 0.0)
    eh1 = t[:, D:] + xs + xd + vec_ref[0:1, :]
    eh1_ref[...] = _pack_cols(eh1)
    mm_ = jnp.where(mask, eh1, 0.0)
    acc[0:1, :] = acc[0:1, :] + jnp.sum(mm_, axis=0, keepdims=True)
    acc[1:2, :] = acc[1:2, :] + jnp.sum(mm_ * mm_, axis=0, keepdims=True)

    @pl.when(i == NBLK - 1)
    def _():
        stats_ref[...] = acc[...]


def _edge1(apply_bn, epre, gsrc, gdst, wcat, vec, pstats):
    ep_w = H // 2 if apply_bn else D
    return pl.pallas_call(
        functools.partial(_edge1_body, apply_bn),
        grid=(NBLK,),
        in_specs=[pl.BlockSpec((BE, ep_w), lambda i: (i, 0)),
                  pl.BlockSpec((BE, D), lambda i: (i, 0)),
                  pl.BlockSpec((BE, H // 2), lambda i: (i, 0)),
                  pl.BlockSpec((D, 2 * D), lambda i: (0, 0)),
                  pl.BlockSpec((8, H), lambda i: (0, 0)),
                  pl.BlockSpec((8, H), lambda i: (0, 0))],
        out_specs=[pl.BlockSpec((BE, D), lambda i: (i, 0)),
                   pl.BlockSpec((BE, H // 2), lambda i: (i, 0)),
                   pl.BlockSpec((8, H), lambda i: (0, 0))],
        out_shape=[jax.ShapeDtypeStruct((EP, D), jnp.float32),
                   jax.ShapeDtypeStruct((EP, H // 2), jnp.uint32),
                   jax.ShapeDtypeStruct((8, H), jnp.float32)],
        scratch_shapes=[pltpu.VMEM((8, H), jnp.float32)],
    )(epre, gsrc, gdst, wcat, vec, pstats)


def _edge2_body(eh1_ref, stats1_ref, vec_ref, w2_ref, eh2_ref, stats2_ref, acc):
    i = pl.program_id(0)

    @pl.when(i == 0)
    def _():
        acc[...] = jnp.zeros((8, H), jnp.float32)

    a = _bn_from_stats(_unpack2(eh1_ref[...]), stats1_ref,
                       vec_ref[0:1, :], vec_ref[1:2, :])
    t = _mmb(a, w2_ref[...]) + vec_ref[2:3, :]
    eh2_ref[...] = _pack_cols(t)
    rows = i * BE + jax.lax.broadcasted_iota(jnp.int32, (BE, 1), 0)
    mask = rows < E
    mm_ = jnp.where(mask, t, 0.0)
    acc[0:1, :] = acc[0:1, :] + jnp.sum(mm_, axis=0, keepdims=True)
    acc[1:2, :] = acc[1:2, :] + jnp.sum(mm_ * mm_, axis=0, keepdims=True)

    @pl.when(i == NBLK - 1)
    def _():
        stats2_ref[...] = acc[...]


def _edge2(eh1, stats1, vec, w2):
    return pl.pallas_call(
        _edge2_body,
        grid=(NBLK,),
        in_specs=[pl.BlockSpec((BE, H // 2), lambda i: (i, 0)),
                  pl.BlockSpec((8, H), lambda i: (0, 0)),
                  pl.BlockSpec((8, H), lambda i: (0, 0)),
                  pl.BlockSpec((H, H), lambda i: (0, 0))],
        out_specs=[pl.BlockSpec((BE, H // 2), lambda i: (i, 0)),
                   pl.BlockSpec((8, H), lambda i: (0, 0))],
        out_shape=[jax.ShapeDtypeStruct((EP, H // 2), jnp.uint32),
                   jax.ShapeDtypeStruct((8, H), jnp.float32)],
        scratch_shapes=[pltpu.VMEM((8, H), jnp.float32)],
    )(eh1, stats1, vec, w2)


def _node_xnew(nagg_ref, x_ref, w1_ref, w2_ref, vec_ref):
    h0 = nagg_ref[...] + x_ref[...]
    y = _mmb(h0, w1_ref[...]) + vec_ref[0:1, :]
    m = jnp.mean(y, axis=0, keepdims=True)
    v = jnp.mean(y * y, axis=0, keepdims=True) - m * m
    h = jnp.maximum(vec_ref[1:2, :] * (y - m) * jax.lax.rsqrt(v + EPS)
                    + vec_ref[2:3, :], 0.0)
    y2 = _mmb(h, w2_ref[...]) + vec_ref[3:4, :]
    m2 = jnp.mean(y2, axis=0, keepdims=True)
    v2 = jnp.mean(y2 * y2, axis=0, keepdims=True) - m2 * m2
    return jnp.maximum(vec_ref[4:5, :] * (y2 - m2)
                       * jax.lax.rsqrt(v2 + EPS) + vec_ref[5:6, :], 0.0)


def _node_body(nagg_ref, x_ref, w1_ref, w2_ref, vec_ref, out_ref):
    out_ref[...] = _node_xnew(nagg_ref, x_ref, w1_ref, w2_ref, vec_ref)


def _node(nagg, x, w1, w2, vec):
    return pl.pallas_call(
        _node_body,
        in_specs=[pl.BlockSpec((N, D), lambda: (0, 0)),
                  pl.BlockSpec((N, D), lambda: (0, 0)),
                  pl.BlockSpec((D, H), lambda: (0, 0)),
                  pl.BlockSpec((H, H), lambda: (0, 0)),
                  pl.BlockSpec((8, H), lambda: (0, 0))],
        out_specs=pl.BlockSpec((N, D), lambda: (0, 0)),
        out_shape=jax.ShapeDtypeStruct((N, D), jnp.float32),
    )(nagg, x, w1, w2, vec)


def _node_fused_body(nagg_ref, x_ref, w1_ref, w2_ref, vec_ref, wcat_ref,
                     bcat_ref, out_ref, tabA_ref, tabB_ref):
    xn = _node_xnew(nagg_ref, x_ref, w1_ref, w2_ref, vec_ref)
    out_ref[...] = xn
    t = _mm(xn, wcat_ref[...]) + bcat_ref[0:1, :]
    tabA_ref[...] = t
    tabB_ref[...] = t[:, D:]


def _node_fused(nagg, x, w1, w2, vec, wcat, bcat):
    return pl.pallas_call(
        _node_fused_body,
        in_specs=[pl.BlockSpec((N, D), lambda: (0, 0)),
                  pl.BlockSpec((N, D), lambda: (0, 0)),
                  pl.BlockSpec((D, H), lambda: (0, 0)),
                  pl.BlockSpec((H, H), lambda: (0, 0)),
                  pl.BlockSpec((8, H), lambda: (0, 0)),
                  pl.BlockSpec((D, 2 * D), lambda: (0, 0)),
                  pl.BlockSpec((8, 2 * D), lambda: (0, 0))],
        out_specs=[pl.BlockSpec((N, D), lambda: (0, 0)),
                   pl.BlockSpec((N, 2 * D), lambda: (0, 0)),
                   pl.BlockSpec((N, H), lambda: (0, 0))],
        out_shape=[jax.ShapeDtypeStruct((N, D), jnp.float32),
                   jax.ShapeDtypeStruct((N, 2 * D), jnp.float32),
                   jax.ShapeDtypeStruct((N, H), jnp.float32)],
    )(nagg, x, w1, w2, vec, wcat, bcat)


def _bnrelu_body(eh2_ref, stats_ref, vec_ref, out_ref):
    i = pl.program_id(0)
    rows = i * BE + jax.lax.broadcasted_iota(jnp.int32, (BE, 1), 0)
    v = _bn_from_stats(_unpack2(eh2_ref[...]), stats_ref,
                       vec_ref[0:1, :], vec_ref[1:2, :])
    out_ref[...] = jnp.where(rows < E, v, 0.0)


def _bnrelu(eh2, stats, vec):
    return pl.pallas_call(
        _bnrelu_body,
        grid=(NBLK,),
        in_specs=[pl.BlockSpec((BE, H // 2), lambda i: (i, 0)),
                  pl.BlockSpec((8, H), lambda i: (0, 0)),
                  pl.BlockSpec((8, H), lambda i: (0, 0))],
        out_specs=pl.BlockSpec((BE, H), lambda i: (i, 0)),
        out_shape=jax.ShapeDtypeStruct((EP, H), jnp.float32),
    )(eh2, stats, vec)


def _pool_body(nblk, limit, width, data_ref, gidx_ref, sum_ref, cnt_ref,
               accs, accc):
    i = pl.program_id(0)

    @pl.when(i == 0)
    def _():
        accs[...] = jnp.zeros((B, width), jnp.float32)
        accc[...] = jnp.zeros((B, 128), jnp.float32)

    d = data_ref[...]
    giota = jax.lax.broadcasted_iota(jnp.int32, (B, 1), 0)
    lane = jax.lax.broadcasted_iota(jnp.int32, (1, 128), 1)
    for r in range(8):
        gr = gidx_ref[r:r + 1, :]
        ids = i * BE + r * 128 + lane
        oh = jnp.where((gr == giota) & (ids < limit), 1.0, 0.0)
        accs[...] = accs[...] + _mm(oh, d[r * 128:(r + 1) * 128, :])
        accc[...] = accc[...] + oh

    @pl.when(i == nblk - 1)
    def _():
        sum_ref[...] = accs[...]
        cnt = jnp.sum(accc[...], axis=1, keepdims=True)
        cnt_ref[...] = jnp.broadcast_to(cnt, (B, 128))


def _pool(limit, data, gidx):
    nblk = data.shape[0] // BE
    width = data.shape[1]
    return pl.pallas_call(
        functools.partial(_pool_body, nblk, limit, width),
        grid=(nblk,),
        in_specs=[pl.BlockSpec((BE, width), lambda i: (i, 0)),
                  pl.BlockSpec((8, 128), lambda i: (i, 0))],
        out_specs=[pl.BlockSpec((B, width), lambda i: (0, 0)),
                   pl.BlockSpec((B, 128), lambda i: (0, 0))],
        out_shape=[jax.ShapeDtypeStruct((B, width), jnp.float32),
                   jax.ShapeDtypeStruct((B, 128), jnp.float32)],
        scratch_shapes=[pltpu.VMEM((B, width), jnp.float32),
                        pltpu.VMEM((B, 128), jnp.float32)],
    )(data, gidx)


def _final_body(pn_ref, cn_ref, pe_ref, ce_ref, w1n_ref, w1e_ref, w2_ref,
                vb_ref, b2_ref, out_ref):
    pn = pn_ref[...] / jnp.maximum(cn_ref[:, 0:1], 1.0)
    pe = pe_ref[...] / jnp.maximum(ce_ref[:, 0:1], 1.0)
    hn = jnp.maximum(_mm(pn, w1n_ref[...]) + vb_ref[0:1, :], 0.0)
    he = jnp.maximum(_mm(pe, w1e_ref[...]) + vb_ref[1:2, :], 0.0)
    out_ref[...] = (_mm(hn, w2_ref[0:FH, :]) + _mm(he, w2_ref[FH:, :])
                    + b2_ref[0:1, :])


def _final(pn, cn, pe, ce, w1n, w1e, w2, vb, b2):
    return pl.pallas_call(
        _final_body,
        in_specs=[pl.BlockSpec((B, H), lambda: (0, 0)),
                  pl.BlockSpec((B, 128), lambda: (0, 0)),
                  pl.BlockSpec((B, H), lambda: (0, 0)),
                  pl.BlockSpec((B, H), lambda: (0, 0)),
                  pl.BlockSpec((H, FH), lambda: (0, 0)),
                  pl.BlockSpec((H, FH), lambda: (0, 0)),
                  pl.BlockSpec((2 * FH, OUT), lambda: (0, 0)),
                  pl.BlockSpec((8, FH), lambda: (0, 0)),
                  pl.BlockSpec((8, OUT), lambda: (0, 0))],
        out_specs=pl.BlockSpec((B, OUT), lambda: (0, 0)),
        out_shape=jax.ShapeDtypeStruct((B, OUT), jnp.float32),
    )(pn, cn, pe, ce, w1n, w1e, w2, vb, b2)


def _vecs8(*rows, width=H):
    out = jnp.zeros((8, width), jnp.float32)
    for r, v in enumerate(rows):
        out = out.at[r, :].set(v)
    return out


def kernel(x, edge_attr, params, edge_index, batch, num_graphs):
    p = params
    src = edge_index[0]
    dst = edge_index[1]
    src_pad = jnp.pad(src, (0, EP - E))
    dst_pad = jnp.pad(dst, (0, EP - E))
    src2d_g = src_pad.reshape(NCH2, CH2)
    dst2d_g = dst_pad.reshape(NCH2, CH2)
    src2d_s = src_pad.reshape(NCH, 128)
    dst2d_s = dst_pad.reshape(NCH, 128)
    epre = jnp.pad(edge_attr, ((0, EP - E), (0, 0)))
    zrows = jnp.zeros((NROW_CP, 128), jnp.float32)
    zstats = jnp.zeros((8, H), jnp.float32)

    def _wcats(l):
        Wx = p["msg_W"][l][:D]
        xcat_W = jnp.concatenate([Wx, p["e1_W"][l]],
                                 axis=1).astype(jnp.bfloat16)
        bcat = _vecs8(jnp.concatenate([p["msg_b"][l], jnp.zeros((H,))]),
                      width=2 * D)
        return xcat_W, bcat

    xcat_W, bcat = _wcats(0)
    tabA, tabB = _node_pre(x, xcat_W, bcat)

    bn_state = None
    for l in range(L):
        We = p["msg_W"][l][D:]
        gsrc, gdst = _gather(tabA, tabB, src2d_g, dst2d_g)

        wcat = jnp.concatenate([We, p["e1_W"][l]],
                               axis=1).astype(jnp.bfloat16)
        if bn_state is None:
            vec1 = _vecs8(p["e1_b"][l])
            msg, eh1, stats1 = _edge1(False, epre, gsrc, gdst, wcat, vec1,
                                      zstats)
        else:
            pstats, pg, pbe = bn_state
            vec1 = _vecs8(p["e1_b"][l], pg, pbe)
            msg, eh1, stats1 = _edge1(True, epre, gsrc, gdst, wcat, vec1,
                                      pstats)

        nagg = _scatter(msg, dst2d_s, zrows)

        vec2 = _vecs8(p["e1_g"][l], p["e1_be"][l], p["e2_b"][l])
        eh2, stats2 = _edge2(eh1, stats1, vec2,
                             p["e2_W"][l].astype(jnp.bfloat16))

        vecn = _vecs8(p["n1_b"][l], p["n1_g"][l], p["n1_be"][l],
                      p["n2_b"][l], p["n2_g"][l], p["n2_be"][l])
        x = _node(nagg, x, p["n1_W"][l].astype(jnp.bfloat16),
                  p["n2_W"][l].astype(jnp.bfloat16), vecn)
        if l < L - 1:
            xcat_W, bcat = _wcats(l + 1)
            tabA, tabB = _node_pre(x, xcat_W, bcat)

        epre = eh2
        bn_state = (stats2, p["e2_g"][l], p["e2_be"][l])

    x_pad = jnp.pad(x, ((0, 10240 - N), (0, 0)))
    batch2d = jnp.pad(batch, (0, 10240 - N), constant_values=B).reshape(80, 128)
    pn_sum, cn = _pool(N, x_pad, batch2d)

    stats2, pg, pbe = bn_state
    efin = _bnrelu(epre, stats2, _vecs8(pg, pbe))
    sn = _scatter(efin, src2d_s, zrows)
    emask = jnp.where(jnp.arange(EP)[:, None] < E, 1.0,
                      0.0).astype(jnp.float32)
    ones_m = jnp.broadcast_to(emask, (EP, D))
    deg = _scatter(ones_m, src2d_s, zrows)
    pe_sum, _ = _pool(N, jnp.pad(sn, ((0, 240, ), (0, 0))), batch2d)
    ce, _ = _pool(N, jnp.pad(deg, ((0, 240), (0, 0))), batch2d)

    vb = _vecs8(p["lin1n_b"], p["lin1e_b"], width=FH)
    b2 = _vecs8(p["lin2_b"], width=OUT)
    out = _final(pn_sum, cn, pe_sum, ce, p["lin1n_W"], p["lin1e_W"],
                 p["lin2_W"], vb, b2)
    return out


# speedup vs baseline: 1.0830x; 1.0422x over previous
import functools

import jax
import jax.numpy as jnp
from jax import lax
from jax.experimental import pallas as pl
from jax.experimental.pallas import tpu as pltpu
from jax.experimental.pallas import tpu_sc as plsc

N = 10000
E = 160000
D = 256
H = 256
L = 4
B = 64
FH = 512
OUT = 128
EPS = 1e-5

EP = 163840
NCH = EP // 128
NW = 32
CHW = NCH // NW
CHT = NCH // 16
BE = 1024
NBLK = EP // BE
NROW_OFF = 624
NROW_CP = 640

_MESH = plsc.VectorSubcoreMesh(core_axis_name="c", subcore_axis_name="s")


def _mm(a, b):
    return jax.lax.dot_general(a, b, (((1,), (0,)), ((), ())),
                               preferred_element_type=jnp.float32)


def _mmb(a, b):
    return jax.lax.dot_general(a.astype(jnp.bfloat16), b,
                               (((1,), (0,)), ((), ())),
                               preferred_element_type=jnp.float32)


CH2 = 128
NCH2 = EP // CH2
CHW_A = 64
CHW_B = NCH2 // 16 - CHW_A


def _gather_body(tabA, tabB, src2d, dst2d, gsrc, gdst, idxs, idxd,
                 bufA, bufB, semG):
    c = lax.axis_index("c")
    s = lax.axis_index("s")
    base = jnp.where(c == 0, s * CHW_A, 16 * CHW_A + s * CHW_B)

    @pl.when(c == 0)
    def _():
        pltpu.sync_copy(src2d.at[pl.ds(s * CHW_A, CHW_A)],
                        idxs.at[pl.ds(0, CHW_A)])
        pltpu.sync_copy(dst2d.at[pl.ds(s * CHW_A, CHW_A)],
                        idxd.at[pl.ds(0, CHW_A)])

    @pl.when(c == 1)
    def _():
        pltpu.sync_copy(src2d.at[pl.ds(16 * CHW_A + s * CHW_B, CHW_B)],
                        idxs.at[pl.ds(0, CHW_B)])
        pltpu.sync_copy(dst2d.at[pl.ds(16 * CHW_A + s * CHW_B, CHW_B)],
                        idxd.at[pl.ds(0, CHW_B)])

    def body(i, _):
        c0 = 2 * i
        c1 = 2 * i + 1
        dA0 = pltpu.async_copy(tabA.at[idxs.at[c0]], bufA.at[0], semG)
        dA1 = pltpu.async_copy(tabA.at[idxs.at[c1]], bufA.at[1], semG)
        dB0 = pltpu.async_copy(tabB.at[idxd.at[c0]], bufB.at[0], semG)
        dB1 = pltpu.async_copy(tabB.at[idxd.at[c1]], bufB.at[1], semG)
        dA0.wait()
        dA1.wait()
        dB0.wait()
        dB1.wait()
        row0 = (base + c0) * CH2
        pltpu.sync_copy(bufA.at[0], gsrc.at[pl.ds(row0, CH2)])
        pltpu.sync_copy(bufA.at[1], gsrc.at[pl.ds(row0 + CH2, CH2)])
        pltpu.sync_copy(bufB.at[0], gdst.at[pl.ds(row0, CH2)])
        pltpu.sync_copy(bufB.at[1], gdst.at[pl.ds(row0 + CH2, CH2)])
        return 0

    npairs = jnp.where(c == 0, CHW_A // 2, CHW_B // 2)
    lax.fori_loop(0, npairs, body, 0)


_gather = pl.kernel(
    _gather_body,
    out_type=[jax.ShapeDtypeStruct((EP, D), jnp.uint32),
              jax.ShapeDtypeStruct((EP, H // 2), jnp.uint32)],
    mesh=_MESH,
    scratch_types=[pltpu.VMEM((CHW_A, CH2), jnp.int32),
                   pltpu.VMEM((CHW_A, CH2), jnp.int32),
                   pltpu.VMEM((2, CH2, D), jnp.uint32),
                   pltpu.VMEM((2, CH2, H // 2), jnp.uint32),
                   pltpu.SemaphoreType.DMA],
)


def _scatter_body(msg, dst2d, zrows, nagg, idxd, buf, accum, semR):
    c = lax.axis_index("c")
    s = lax.axis_index("s")
    coff = c * 128
    pltpu.sync_copy(zrows, accum.at[pl.ds(s * NROW_OFF, NROW_CP)])
    plsc.subcore_barrier()
    pltpu.sync_copy(dst2d.at[pl.ds(s * CHT, CHT)], idxd)

    def body(i, _):
        c0 = 2 * i
        c1 = 2 * i + 1
        row0 = s * (CHT * 128) + c0 * 128
        d0 = pltpu.async_copy(msg.at[pl.ds(row0, 128), pl.ds(coff, 128)],
                              buf.at[0], semR)
        d1 = pltpu.async_copy(msg.at[pl.ds(row0 + 128, 128),
                                     pl.ds(coff, 128)], buf.at[1], semR)
        d0.wait()
        d1.wait()
        pltpu.sync_copy(buf.at[0], accum.at[idxd.at[c0]], add=True)
        pltpu.sync_copy(buf.at[1], accum.at[idxd.at[c1]], add=True)
        return 0

    lax.fori_loop(0, CHT // 2, body, 0)
    plsc.subcore_barrier()
    pltpu.sync_copy(accum.at[pl.ds(s * NROW_OFF, NROW_CP)],
                    nagg.at[pl.ds(s * NROW_OFF, NROW_CP), pl.ds(coff, 128)])


_scatter = pl.kernel(
    _scatter_body,
    out_type=jax.ShapeDtypeStruct((N, D), jnp.float32),
    mesh=_MESH,
    scratch_types=[pltpu.VMEM((CHT, 128), jnp.int32),
                   pltpu.VMEM((2, 128, 128), jnp.float32),
                   pltpu.MemorySpace.VMEM_SHARED((N, 128), jnp.float32),
                   pltpu.SemaphoreType.DMA],
)


def _scatter_ones_body(ones128, dst2d, zrows, nagg, idxd, buf, accum):
    c = lax.axis_index("c")
    s = lax.axis_index("s")
    coff = c * 128
    pltpu.sync_copy(zrows, accum.at[pl.ds(s * NROW_OFF, NROW_CP)])
    plsc.subcore_barrier()
    pltpu.sync_copy(dst2d.at[pl.ds(s * CHT, CHT)], idxd)
    pltpu.sync_copy(ones128, buf)

    def body(ci, _):
        @pl.when(s * CHT + ci < E // 128)
        def _():
            pltpu.sync_copy(buf, accum.at[idxd.at[ci]], add=True)

        return 0

    lax.fori_loop(0, CHT, body, 0)
    plsc.subcore_barrier()
    pltpu.sync_copy(accum.at[pl.ds(s * NROW_OFF, NROW_CP)],
                    nagg.at[pl.ds(s * NROW_OFF, NROW_CP), pl.ds(coff, 128)])


_scatter_ones = pl.kernel(
    _scatter_ones_body,
    out_type=jax.ShapeDtypeStruct((N, D), jnp.float32),
    mesh=_MESH,
    scratch_types=[pltpu.VMEM((CHT, 128), jnp.int32),
                   pltpu.VMEM((128, 128), jnp.float32),
                   pltpu.MemorySpace.VMEM_SHARED((N, 128), jnp.float32)],
)


def _rb(x):
    return jax.lax.bitcast_convert_type(
        x.astype(jnp.bfloat16).astype(jnp.float32), jnp.uint32)


def _pack2(hi, lo):
    return jax.lax.bitwise_or(
        _rb(hi), jax.lax.shift_right_logical(_rb(lo), jnp.uint32(16)))


def _lo_f32(pk):
    return jax.lax.bitcast_convert_type(
        jax.lax.shift_left(pk, jnp.uint32(16)), jnp.float32)


def _hi_f32(pk):
    return jax.lax.bitcast_convert_type(
        jax.lax.bitwise_and(pk, jnp.uint32(0xFFFF0000)), jnp.float32)


def _node_pre_body(x_ref, w_ref, b_ref, tabA_ref, tabB_ref):
    t = _mmb(x_ref[...], w_ref[...]) + b_ref[0:1, :]
    xm = t[:, :D]
    xe = t[:, D:]
    tabA_ref[...] = _pack2(xe, xm)
    tabB_ref[...] = _pack2(xe[:, H // 2:], xe[:, :H // 2])


def _node_pre(x, wcat, bcat):
    return pl.pallas_call(
        _node_pre_body,
        grid=(5,),
        in_specs=[pl.BlockSpec((2000, D), lambda i: (i, 0)),
                  pl.BlockSpec((D, 2 * D), lambda i: (0, 0)),
                  pl.BlockSpec((8, 2 * D), lambda i: (0, 0))],
        out_specs=[pl.BlockSpec((2000, D), lambda i: (i, 0)),
                   pl.BlockSpec((2000, H // 2), lambda i: (i, 0))],
        out_shape=[jax.ShapeDtypeStruct((N, D), jnp.uint32),
                   jax.ShapeDtypeStruct((N, H // 2), jnp.uint32)],
    )(x, wcat, bcat)


def _bn_from_stats(t, stats_ref, g, be):
    m = stats_ref[0:1, :] / E
    v = stats_ref[1:2, :] / E - m * m
    inv = jax.lax.rsqrt(v + EPS)
    return jnp.maximum(g * (t - m) * inv + be, 0.0)


def _unpack2(pk):
    return jnp.concatenate([_lo_f32(pk), _hi_f32(pk)], axis=1)


def _pack_cols(t):
    w = t.shape[1] // 2
    return _pack2(t[:, w:], t[:, :w])


def _edge1_body(apply_bn, epre_ref, gsrc_ref, gdst_ref, wcat_ref, vec_ref,
                pstats_ref, msg_ref, eh1_ref, stats_ref, acc):
    i = pl.program_id(0)

    @pl.when(i == 0)
    def _():
        acc[...] = jnp.zeros((8, H), jnp.float32)

    if apply_bn:
        ep = _unpack2(epre_ref[...])
        ebn = _bn_from_stats(ep, pstats_ref, vec_ref[1:2, :], vec_ref[2:3, :])
    else:
        ebn = epre_ref[...]
    t = _mmb(ebn, wcat_ref[...])
    gp = gsrc_ref[...]
    dp = gdst_ref[...]
    xm = _lo_f32(gp)
    xs = _hi_f32(gp)
    xd = jnp.concatenate([_lo_f32(dp), _hi_f32(dp)], axis=1)
    rows = i * BE + jax.lax.broadcasted_iota(jnp.int32, (BE, 1), 0)
    mask = rows < E
    msg = jnp.maximum(t[:, :D] + xm, 0.0)
    msg_ref[...] = jnp.where(mask, msg, 0.0)
    eh1 = t[:, D:] + xs + xd + vec_ref[0:1, :]
    eh1_ref[...] = _pack_cols(eh1)
    mm_ = jnp.where(mask, eh1, 0.0)
    acc[0:1, :] = acc[0:1, :] + jnp.sum(mm_, axis=0, keepdims=True)
    acc[1:2, :] = acc[1:2, :] + jnp.sum(mm_ * mm_, axis=0, keepdims=True)

    @pl.when(i == NBLK - 1)
    def _():
        stats_ref[...] = acc[...]


def _edge1(apply_bn, epre, gsrc, gdst, wcat, vec, pstats):
    ep_w = H // 2 if apply_bn else D
    return pl.pallas_call(
        functools.partial(_edge1_body, apply_bn),
        grid=(NBLK,),
        in_specs=[pl.BlockSpec((BE, ep_w), lambda i: (i, 0)),
                  pl.BlockSpec((BE, D), lambda i: (i, 0)),
                  pl.BlockSpec((BE, H // 2), lambda i: (i, 0)),
                  pl.BlockSpec((D, 2 * D), lambda i: (0, 0)),
                  pl.BlockSpec((8, H), lambda i: (0, 0)),
                  pl.BlockSpec((8, H), lambda i: (0, 0))],
        out_specs=[pl.BlockSpec((BE, D), lambda i: (i, 0)),
                   pl.BlockSpec((BE, H // 2), lambda i: (i, 0)),
                   pl.BlockSpec((8, H), lambda i: (0, 0))],
        out_shape=[jax.ShapeDtypeStruct((EP, D), jnp.float32),
                   jax.ShapeDtypeStruct((EP, H // 2), jnp.uint32),
                   jax.ShapeDtypeStruct((8, H), jnp.float32)],
        scratch_shapes=[pltpu.VMEM((8, H), jnp.float32)],
    )(epre, gsrc, gdst, wcat, vec, pstats)


def _edge2_body(eh1_ref, stats1_ref, vec_ref, w2_ref, eh2_ref, stats2_ref, acc):
    i = pl.program_id(0)

    @pl.when(i == 0)
    def _():
        acc[...] = jnp.zeros((8, H), jnp.float32)

    a = _bn_from_stats(_unpack2(eh1_ref[...]), stats1_ref,
                       vec_ref[0:1, :], vec_ref[1:2, :])
    t = _mmb(a, w2_ref[...]) + vec_ref[2:3, :]
    eh2_ref[...] = _pack_cols(t)
    rows = i * BE + jax.lax.broadcasted_iota(jnp.int32, (BE, 1), 0)
    mask = rows < E
    mm_ = jnp.where(mask, t, 0.0)
    acc[0:1, :] = acc[0:1, :] + jnp.sum(mm_, axis=0, keepdims=True)
    acc[1:2, :] = acc[1:2, :] + jnp.sum(mm_ * mm_, axis=0, keepdims=True)

    @pl.when(i == NBLK - 1)
    def _():
        stats2_ref[...] = acc[...]


def _edge2(eh1, stats1, vec, w2):
    return pl.pallas_call(
        _edge2_body,
        grid=(NBLK,),
        in_specs=[pl.BlockSpec((BE, H // 2), lambda i: (i, 0)),
                  pl.BlockSpec((8, H), lambda i: (0, 0)),
                  pl.BlockSpec((8, H), lambda i: (0, 0)),
                  pl.BlockSpec((H, H), lambda i: (0, 0))],
        out_specs=[pl.BlockSpec((BE, H // 2), lambda i: (i, 0)),
                   pl.BlockSpec((8, H), lambda i: (0, 0))],
        out_shape=[jax.ShapeDtypeStruct((EP, H // 2), jnp.uint32),
                   jax.ShapeDtypeStruct((8, H), jnp.float32)],
        scratch_shapes=[pltpu.VMEM((8, H), jnp.float32)],
    )(eh1, stats1, vec, w2)


def _node_xnew(nagg_ref, x_ref, w1_ref, w2_ref, vec_ref):
    h0 = nagg_ref[...] + x_ref[...]
    y = _mmb(h0, w1_ref[...]) + vec_ref[0:1, :]
    m = jnp.mean(y, axis=0, keepdims=True)
    v = jnp.mean(y * y, axis=0, keepdims=True) - m * m
    h = jnp.maximum(vec_ref[1:2, :] * (y - m) * jax.lax.rsqrt(v + EPS)
                    + vec_ref[2:3, :], 0.0)
    y2 = _mmb(h, w2_ref[...]) + vec_ref[3:4, :]
    m2 = jnp.mean(y2, axis=0, keepdims=True)
    v2 = jnp.mean(y2 * y2, axis=0, keepdims=True) - m2 * m2
    return jnp.maximum(vec_ref[4:5, :] * (y2 - m2)
                       * jax.lax.rsqrt(v2 + EPS) + vec_ref[5:6, :], 0.0)


def _node_body(nagg_ref, x_ref, w1_ref, w2_ref, vec_ref, out_ref):
    out_ref[...] = _node_xnew(nagg_ref, x_ref, w1_ref, w2_ref, vec_ref)


def _node(nagg, x, w1, w2, vec):
    return pl.pallas_call(
        _node_body,
        in_specs=[pl.BlockSpec((N, D), lambda: (0, 0)),
                  pl.BlockSpec((N, D), lambda: (0, 0)),
                  pl.BlockSpec((D, H), lambda: (0, 0)),
                  pl.BlockSpec((H, H), lambda: (0, 0)),
                  pl.BlockSpec((8, H), lambda: (0, 0))],
        out_specs=pl.BlockSpec((N, D), lambda: (0, 0)),
        out_shape=jax.ShapeDtypeStruct((N, D), jnp.float32),
    )(nagg, x, w1, w2, vec)


def _node_fused_body(nagg_ref, x_ref, w1_ref, w2_ref, vec_ref, wcat_ref,
                     bcat_ref, out_ref, tabA_ref, tabB_ref):
    xn = _node_xnew(nagg_ref, x_ref, w1_ref, w2_ref, vec_ref)
    out_ref[...] = xn
    t = _mm(xn, wcat_ref[...]) + bcat_ref[0:1, :]
    tabA_ref[...] = t
    tabB_ref[...] = t[:, D:]


def _node_fused(nagg, x, w1, w2, vec, wcat, bcat):
    return pl.pallas_call(
        _node_fused_body,
        in_specs=[pl.BlockSpec((N, D), lambda: (0, 0)),
                  pl.BlockSpec((N, D), lambda: (0, 0)),
                  pl.BlockSpec((D, H), lambda: (0, 0)),
                  pl.BlockSpec((H, H), lambda: (0, 0)),
                  pl.BlockSpec((8, H), lambda: (0, 0)),
                  pl.BlockSpec((D, 2 * D), lambda: (0, 0)),
                  pl.BlockSpec((8, 2 * D), lambda: (0, 0))],
        out_specs=[pl.BlockSpec((N, D), lambda: (0, 0)),
                   pl.BlockSpec((N, 2 * D), lambda: (0, 0)),
                   pl.BlockSpec((N, H), lambda: (0, 0))],
        out_shape=[jax.ShapeDtypeStruct((N, D), jnp.float32),
                   jax.ShapeDtypeStruct((N, 2 * D), jnp.float32),
                   jax.ShapeDtypeStruct((N, H), jnp.float32)],
    )(nagg, x, w1, w2, vec, wcat, bcat)


def _bnrelu_body(eh2_ref, stats_ref, vec_ref, out_ref):
    i = pl.program_id(0)
    rows = i * BE + jax.lax.broadcasted_iota(jnp.int32, (BE, 1), 0)
    v = _bn_from_stats(_unpack2(eh2_ref[...]), stats_ref,
                       vec_ref[0:1, :], vec_ref[1:2, :])
    out_ref[...] = jnp.where(rows < E, v, 0.0)


def _bnrelu(eh2, stats, vec):
    return pl.pallas_call(
        _bnrelu_body,
        grid=(NBLK,),
        in_specs=[pl.BlockSpec((BE, H // 2), lambda i: (i, 0)),
                  pl.BlockSpec((8, H), lambda i: (0, 0)),
                  pl.BlockSpec((8, H), lambda i: (0, 0))],
        out_specs=pl.BlockSpec((BE, H), lambda i: (i, 0)),
        out_shape=jax.ShapeDtypeStruct((EP, H), jnp.float32),
    )(eh2, stats, vec)


def _pool_body(nblk, limit, width, data_ref, gidx_ref, sum_ref, cnt_ref,
               accs, accc):
    i = pl.program_id(0)

    @pl.when(i == 0)
    def _():
        accs[...] = jnp.zeros((B, width), jnp.float32)
        accc[...] = jnp.zeros((B, 128), jnp.float32)

    d = data_ref[...]
    giota = jax.lax.broadcasted_iota(jnp.int32, (B, 1), 0)
    lane = jax.lax.broadcasted_iota(jnp.int32, (1, 128), 1)
    for r in range(8):
        gr = gidx_ref[r:r + 1, :]
        ids = i * BE + r * 128 + lane
        oh = jnp.where((gr == giota) & (ids < limit), 1.0, 0.0)
        accs[...] = accs[...] + _mm(oh, d[r * 128:(r + 1) * 128, :])
        accc[...] = accc[...] + oh

    @pl.when(i == nblk - 1)
    def _():
        sum_ref[...] = accs[...]
        cnt = jnp.sum(accc[...], axis=1, keepdims=True)
        cnt_ref[...] = jnp.broadcast_to(cnt, (B, 128))


def _pool(limit, data, gidx):
    nblk = data.shape[0] // BE
    width = data.shape[1]
    return pl.pallas_call(
        functools.partial(_pool_body, nblk, limit, width),
        grid=(nblk,),
        in_specs=[pl.BlockSpec((BE, width), lambda i: (i, 0)),
                  pl.BlockSpec((8, 128), lambda i: (i, 0))],
        out_specs=[pl.BlockSpec((B, width), lambda i: (0, 0)),
                   pl.BlockSpec((B, 128), lambda i: (0, 0))],
        out_shape=[jax.ShapeDtypeStruct((B, width), jnp.float32),
                   jax.ShapeDtypeStruct((B, 128), jnp.float32)],
        scratch_shapes=[pltpu.VMEM((B, width), jnp.float32),
                        pltpu.VMEM((B, 128), jnp.float32)],
    )(data, gidx)


def _final_body(pn_ref, cn_ref, pe_ref, ce_ref, w1n_ref, w1e_ref, w2_ref,
                vb_ref, b2_ref, out_ref):
    pn = pn_ref[...] / jnp.maximum(cn_ref[:, 0:1], 1.0)
    pe = pe_ref[...] / jnp.maximum(ce_ref[:, 0:1], 1.0)
    hn = jnp.maximum(_mm(pn, w1n_ref[...]) + vb_ref[0:1, :], 0.0)
    he = jnp.maximum(_mm(pe, w1e_ref[...]) + vb_ref[1:2, :], 0.0)
    out_ref[...] = (_mm(hn, w2_ref[0:FH, :]) + _mm(he, w2_ref[FH:, :])
                    + b2_ref[0:1, :])


def _final(pn, cn, pe, ce, w1n, w1e, w2, vb, b2):
    return pl.pallas_call(
        _final_body,
        in_specs=[pl.BlockSpec((B, H), lambda: (0, 0)),
                  pl.BlockSpec((B, 128), lambda: (0, 0)),
                  pl.BlockSpec((B, H), lambda: (0, 0)),
                  pl.BlockSpec((B, H), lambda: (0, 0)),
                  pl.BlockSpec((H, FH), lambda: (0, 0)),
                  pl.BlockSpec((H, FH), lambda: (0, 0)),
                  pl.BlockSpec((2 * FH, OUT), lambda: (0, 0)),
                  pl.BlockSpec((8, FH), lambda: (0, 0)),
                  pl.BlockSpec((8, OUT), lambda: (0, 0))],
        out_specs=pl.BlockSpec((B, OUT), lambda: (0, 0)),
        out_shape=jax.ShapeDtypeStruct((B, OUT), jnp.float32),
    )(pn, cn, pe, ce, w1n, w1e, w2, vb, b2)


def _vecs8(*rows, width=H):
    out = jnp.zeros((8, width), jnp.float32)
    for r, v in enumerate(rows):
        out = out.at[r, :].set(v)
    return out


def kernel(x, edge_attr, params, edge_index, batch, num_graphs):
    p = params
    src = edge_index[0]
    dst = edge_index[1]
    src_pad = jnp.pad(src, (0, EP - E))
    dst_pad = jnp.pad(dst, (0, EP - E))
    src2d_g = src_pad.reshape(NCH2, CH2)
    dst2d_g = dst_pad.reshape(NCH2, CH2)
    src2d_s = src_pad.reshape(NCH, 128)
    dst2d_s = dst_pad.reshape(NCH, 128)
    epre = jnp.pad(edge_attr, ((0, EP - E), (0, 0)))
    zrows = jnp.zeros((NROW_CP, 128), jnp.float32)
    zstats = jnp.zeros((8, H), jnp.float32)

    def _wcats(l):
        Wx = p["msg_W"][l][:D]
        xcat_W = jnp.concatenate([Wx, p["e1_W"][l]],
                                 axis=1).astype(jnp.bfloat16)
        bcat = _vecs8(jnp.concatenate([p["msg_b"][l], jnp.zeros((H,))]),
                      width=2 * D)
        return xcat_W, bcat

    xcat_W, bcat = _wcats(0)
    tabA, tabB = _node_pre(x, xcat_W, bcat)

    bn_state = None
    for l in range(L):
        We = p["msg_W"][l][D:]
        gsrc, gdst = _gather(tabA, tabB, src2d_g, dst2d_g)

        wcat = jnp.concatenate([We, p["e1_W"][l]],
                               axis=1).astype(jnp.bfloat16)
        if bn_state is None:
            vec1 = _vecs8(p["e1_b"][l])
            msg, eh1, stats1 = _edge1(False, epre, gsrc, gdst, wcat, vec1,
                                      zstats)
        else:
            pstats, pg, pbe = bn_state
            vec1 = _vecs8(p["e1_b"][l], pg, pbe)
            msg, eh1, stats1 = _edge1(True, epre, gsrc, gdst, wcat, vec1,
                                      pstats)

        nagg = _scatter(msg, dst2d_s, zrows)

        vec2 = _vecs8(p["e1_g"][l], p["e1_be"][l], p["e2_b"][l])
        eh2, stats2 = _edge2(eh1, stats1, vec2,
                             p["e2_W"][l].astype(jnp.bfloat16))

        vecn = _vecs8(p["n1_b"][l], p["n1_g"][l], p["n1_be"][l],
                      p["n2_b"][l], p["n2_g"][l], p["n2_be"][l])
        x = _node(nagg, x, p["n1_W"][l].astype(jnp.bfloat16),
                  p["n2_W"][l].astype(jnp.bfloat16), vecn)
        if l < L - 1:
            xcat_W, bcat = _wcats(l + 1)
            tabA, tabB = _node_pre(x, xcat_W, bcat)

        epre = eh2
        bn_state = (stats2, p["e2_g"][l], p["e2_be"][l])

    x_pad = jnp.pad(x, ((0, 10240 - N), (0, 0)))
    batch2d = jnp.pad(batch, (0, 10240 - N), constant_values=B).reshape(80, 128)
    pn_sum, cn = _pool(N, x_pad, batch2d)

    stats2, pg, pbe = bn_state
    efin = _bnrelu(epre, stats2, _vecs8(pg, pbe))
    sn = _scatter(efin, src2d_s, zrows)
    deg = _scatter_ones(jnp.ones((128, 128), jnp.float32), src2d_s, zrows)
    pe_sum, _ = _pool(N, jnp.pad(sn, ((0, 240, ), (0, 0))), batch2d)
    ce, _ = _pool(N, jnp.pad(deg, ((0, 240), (0, 0))), batch2d)

    vb = _vecs8(p["lin1n_b"], p["lin1e_b"], width=FH)
    b2 = _vecs8(p["lin2_b"], width=OUT)
    out = _final(pn_sum, cn, pe_sum, ce, p["lin1n_W"], p["lin1e_W"],
                 p["lin2_W"], vb, b2)
    return out


# speedup vs baseline: 1.1564x; 1.0677x over previous
import functools

import jax
import jax.numpy as jnp
from jax import lax
from jax.experimental import pallas as pl
from jax.experimental.pallas import tpu as pltpu
from jax.experimental.pallas import tpu_sc as plsc

N = 10000
E = 160000
D = 256
H = 256
L = 4
B = 64
FH = 512
OUT = 128
EPS = 1e-5

EP = 163840
NCH = EP // 128
NW = 32
CHW = NCH // NW
CHT = NCH // 16
BE = 2048
NBLK = EP // BE
NROW_OFF = 624
NROW_CP = 640

_MESH = plsc.VectorSubcoreMesh(core_axis_name="c", subcore_axis_name="s")


def _mm(a, b):
    return jax.lax.dot_general(a, b, (((1,), (0,)), ((), ())),
                               preferred_element_type=jnp.float32)


def _mmb(a, b):
    return jax.lax.dot_general(a.astype(jnp.bfloat16), b,
                               (((1,), (0,)), ((), ())),
                               preferred_element_type=jnp.float32)


CH2 = 128
NCH2 = EP // CH2
CHW_A = 64
CHW_B = NCH2 // 16 - CHW_A


def _gather_body(tabA, tabB, src2d, dst2d, gsrc, gdst, idxs, idxd,
                 bufA, bufB, semG):
    c = lax.axis_index("c")
    s = lax.axis_index("s")
    base = jnp.where(c == 0, s * CHW_A, 16 * CHW_A + s * CHW_B)

    @pl.when(c == 0)
    def _():
        pltpu.sync_copy(src2d.at[pl.ds(s * CHW_A, CHW_A)],
                        idxs.at[pl.ds(0, CHW_A)])
        pltpu.sync_copy(dst2d.at[pl.ds(s * CHW_A, CHW_A)],
                        idxd.at[pl.ds(0, CHW_A)])

    @pl.when(c == 1)
    def _():
        pltpu.sync_copy(src2d.at[pl.ds(16 * CHW_A + s * CHW_B, CHW_B)],
                        idxs.at[pl.ds(0, CHW_B)])
        pltpu.sync_copy(dst2d.at[pl.ds(16 * CHW_A + s * CHW_B, CHW_B)],
                        idxd.at[pl.ds(0, CHW_B)])

    def body(i, _):
        c0 = 2 * i
        c1 = 2 * i + 1
        dA0 = pltpu.async_copy(tabA.at[idxs.at[c0]], bufA.at[0], semG)
        dA1 = pltpu.async_copy(tabA.at[idxs.at[c1]], bufA.at[1], semG)
        dB0 = pltpu.async_copy(tabB.at[idxd.at[c0]], bufB.at[0], semG)
        dB1 = pltpu.async_copy(tabB.at[idxd.at[c1]], bufB.at[1], semG)
        dA0.wait()
        dA1.wait()
        dB0.wait()
        dB1.wait()
        row0 = (base + c0) * CH2
        pltpu.sync_copy(bufA.at[0], gsrc.at[pl.ds(row0, CH2)])
        pltpu.sync_copy(bufA.at[1], gsrc.at[pl.ds(row0 + CH2, CH2)])
        pltpu.sync_copy(bufB.at[0], gdst.at[pl.ds(row0, CH2)])
        pltpu.sync_copy(bufB.at[1], gdst.at[pl.ds(row0 + CH2, CH2)])
        return 0

    npairs = jnp.where(c == 0, CHW_A // 2, CHW_B // 2)
    lax.fori_loop(0, npairs, body, 0)


_gather = pl.kernel(
    _gather_body,
    out_type=[jax.ShapeDtypeStruct((EP, D), jnp.uint32),
              jax.ShapeDtypeStruct((EP, H // 2), jnp.uint32)],
    mesh=_MESH,
    scratch_types=[pltpu.VMEM((CHW_A, CH2), jnp.int32),
                   pltpu.VMEM((CHW_A, CH2), jnp.int32),
                   pltpu.VMEM((2, CH2, D), jnp.uint32),
                   pltpu.VMEM((2, CH2, H // 2), jnp.uint32),
                   pltpu.SemaphoreType.DMA],
)


def _scatter_body(msg, dst2d, zrows, nagg, idxd, buf, accum, semR):
    c = lax.axis_index("c")
    s = lax.axis_index("s")
    coff = c * 128
    pltpu.sync_copy(zrows, accum.at[pl.ds(s * NROW_OFF, NROW_CP)])
    plsc.subcore_barrier()
    pltpu.sync_copy(dst2d.at[pl.ds(s * CHT, CHT)], idxd)

    def body(i, _):
        c0 = 2 * i
        c1 = 2 * i + 1
        row0 = s * (CHT * 128) + c0 * 128
        d0 = pltpu.async_copy(msg.at[pl.ds(row0, 128), pl.ds(coff, 128)],
                              buf.at[0], semR)
        d1 = pltpu.async_copy(msg.at[pl.ds(row0 + 128, 128),
                                     pl.ds(coff, 128)], buf.at[1], semR)
        d0.wait()
        d1.wait()
        pltpu.sync_copy(buf.at[0], accum.at[idxd.at[c0]], add=True)
        pltpu.sync_copy(buf.at[1], accum.at[idxd.at[c1]], add=True)
        return 0

    lax.fori_loop(0, CHT // 2, body, 0)
    plsc.subcore_barrier()
    pltpu.sync_copy(accum.at[pl.ds(s * NROW_OFF, NROW_CP)],
                    nagg.at[pl.ds(s * NROW_OFF, NROW_CP), pl.ds(coff, 128)])


_scatter = pl.kernel(
    _scatter_body,
    out_type=jax.ShapeDtypeStruct((N, D), jnp.float32),
    mesh=_MESH,
    scratch_types=[pltpu.VMEM((CHT, 128), jnp.int32),
                   pltpu.VMEM((2, 128, 128), jnp.float32),
                   pltpu.MemorySpace.VMEM_SHARED((N, 128), jnp.float32),
                   pltpu.SemaphoreType.DMA],
)


def _scatter_ones_body(ones128, dst2d, zrows, nagg, idxd, buf, accum):
    c = lax.axis_index("c")
    s = lax.axis_index("s")
    coff = c * 128
    pltpu.sync_copy(zrows, accum.at[pl.ds(s * NROW_OFF, NROW_CP)])
    plsc.subcore_barrier()
    pltpu.sync_copy(dst2d.at[pl.ds(s * CHT, CHT)], idxd)
    pltpu.sync_copy(ones128, buf)

    def body(ci, _):
        @pl.when(s * CHT + ci < E // 128)
        def _():
            pltpu.sync_copy(buf, accum.at[idxd.at[ci]], add=True)

        return 0

    lax.fori_loop(0, CHT, body, 0)
    plsc.subcore_barrier()
    pltpu.sync_copy(accum.at[pl.ds(s * NROW_OFF, NROW_CP)],
                    nagg.at[pl.ds(s * NROW_OFF, NROW_CP), pl.ds(coff, 128)])


_scatter_ones = pl.kernel(
    _scatter_ones_body,
    out_type=jax.ShapeDtypeStruct((N, D), jnp.float32),
    mesh=_MESH,
    scratch_types=[pltpu.VMEM((CHT, 128), jnp.int32),
                   pltpu.VMEM((128, 128), jnp.float32),
                   pltpu.MemorySpace.VMEM_SHARED((N, 128), jnp.float32)],
)


def _rb(x):
    return jax.lax.bitcast_convert_type(
        x.astype(jnp.bfloat16).astype(jnp.float32), jnp.uint32)


def _pack2(hi, lo):
    return jax.lax.bitwise_or(
        _rb(hi), jax.lax.shift_right_logical(_rb(lo), jnp.uint32(16)))


def _lo_f32(pk):
    return jax.lax.bitcast_convert_type(
        jax.lax.shift_left(pk, jnp.uint32(16)), jnp.float32)


def _hi_f32(pk):
    return jax.lax.bitcast_convert_type(
        jax.lax.bitwise_and(pk, jnp.uint32(0xFFFF0000)), jnp.float32)


def _node_pre_body(x_ref, w_ref, b_ref, tabA_ref, tabB_ref):
    t = _mmb(x_ref[...], w_ref[...]) + b_ref[0:1, :]
    xm = t[:, :D]
    xe = t[:, D:]
    tabA_ref[...] = _pack2(xe, xm)
    tabB_ref[...] = _pack2(xe[:, H // 2:], xe[:, :H // 2])


def _node_pre(x, wcat, bcat):
    return pl.pallas_call(
        _node_pre_body,
        grid=(5,),
        in_specs=[pl.BlockSpec((2000, D), lambda i: (i, 0)),
                  pl.BlockSpec((D, 2 * D), lambda i: (0, 0)),
                  pl.BlockSpec((8, 2 * D), lambda i: (0, 0))],
        out_specs=[pl.BlockSpec((2000, D), lambda i: (i, 0)),
                   pl.BlockSpec((2000, H // 2), lambda i: (i, 0))],
        out_shape=[jax.ShapeDtypeStruct((N, D), jnp.uint32),
                   jax.ShapeDtypeStruct((N, H // 2), jnp.uint32)],
    )(x, wcat, bcat)


def _bn_from_stats(t, stats_ref, g, be):
    m = stats_ref[0:1, :] / E
    v = stats_ref[1:2, :] / E - m * m
    inv = jax.lax.rsqrt(v + EPS)
    return jnp.maximum(g * (t - m) * inv + be, 0.0)


def _unpack2(pk):
    return jnp.concatenate([_lo_f32(pk), _hi_f32(pk)], axis=1)


def _pack_cols(t):
    w = t.shape[1] // 2
    return _pack2(t[:, w:], t[:, :w])


def _edge1_body(apply_bn, epre_ref, gsrc_ref, gdst_ref, wcat_ref, vec_ref,
                pstats_ref, msg_ref, eh1_ref, stats_ref, acc):
    i = pl.program_id(0)

    @pl.when(i == 0)
    def _():
        acc[...] = jnp.zeros((8, H), jnp.float32)

    if apply_bn:
        ep = _unpack2(epre_ref[...])
        ebn = _bn_from_stats(ep, pstats_ref, vec_ref[1:2, :], vec_ref[2:3, :])
    else:
        ebn = epre_ref[...]
    t = _mmb(ebn, wcat_ref[...])
    gp = gsrc_ref[...]
    dp = gdst_ref[...]
    xm = _lo_f32(gp)
    xs = _hi_f32(gp)
    xd = jnp.concatenate([_lo_f32(dp), _hi_f32(dp)], axis=1)
    rows = i * BE + jax.lax.broadcasted_iota(jnp.int32, (BE, 1), 0)
    mask = rows < E
    msg = jnp.maximum(t[:, :D] + xm, 0.0)
    msg_ref[...] = jnp.where(mask, msg, 0.0)
    eh1 = t[:, D:] + xs + xd + vec_ref[0:1, :]
    eh1_ref[...] = _pack_cols(eh1)
    mm_ = jnp.where(mask, eh1, 0.0)
    acc[0:1, :] = acc[0:1, :] + jnp.sum(mm_, axis=0, keepdims=True)
    acc[1:2, :] = acc[1:2, :] + jnp.sum(mm_ * mm_, axis=0, keepdims=True)

    @pl.when(i == NBLK - 1)
    def _():
        stats_ref[...] = acc[...]


def _edge1(apply_bn, epre, gsrc, gdst, wcat, vec, pstats):
    ep_w = H // 2 if apply_bn else D
    return pl.pallas_call(
        functools.partial(_edge1_body, apply_bn),
        grid=(NBLK,),
        in_specs=[pl.BlockSpec((BE, ep_w), lambda i: (i, 0)),
                  pl.BlockSpec((BE, D), lambda i: (i, 0)),
                  pl.BlockSpec((BE, H // 2), lambda i: (i, 0)),
                  pl.BlockSpec((D, 2 * D), lambda i: (0, 0)),
                  pl.BlockSpec((8, H), lambda i: (0, 0)),
                  pl.BlockSpec((8, H), lambda i: (0, 0))],
        out_specs=[pl.BlockSpec((BE, D), lambda i: (i, 0)),
                   pl.BlockSpec((BE, H // 2), lambda i: (i, 0)),
                   pl.BlockSpec((8, H), lambda i: (0, 0))],
        out_shape=[jax.ShapeDtypeStruct((EP, D), jnp.float32),
                   jax.ShapeDtypeStruct((EP, H // 2), jnp.uint32),
                   jax.ShapeDtypeStruct((8, H), jnp.float32)],
        scratch_shapes=[pltpu.VMEM((8, H), jnp.float32)],
    )(epre, gsrc, gdst, wcat, vec, pstats)


def _edge2_body(eh1_ref, stats1_ref, vec_ref, w2_ref, eh2_ref, stats2_ref, acc):
    i = pl.program_id(0)

    @pl.when(i == 0)
    def _():
        acc[...] = jnp.zeros((8, H), jnp.float32)

    a = _bn_from_stats(_unpack2(eh1_ref[...]), stats1_ref,
                       vec_ref[0:1, :], vec_ref[1:2, :])
    t = _mmb(a, w2_ref[...]) + vec_ref[2:3, :]
    eh2_ref[...] = _pack_cols(t)
    rows = i * BE + jax.lax.broadcasted_iota(jnp.int32, (BE, 1), 0)
    mask = rows < E
    mm_ = jnp.where(mask, t, 0.0)
    acc[0:1, :] = acc[0:1, :] + jnp.sum(mm_, axis=0, keepdims=True)
    acc[1:2, :] = acc[1:2, :] + jnp.sum(mm_ * mm_, axis=0, keepdims=True)

    @pl.when(i == NBLK - 1)
    def _():
        stats2_ref[...] = acc[...]


def _edge2(eh1, stats1, vec, w2):
    return pl.pallas_call(
        _edge2_body,
        grid=(NBLK,),
        in_specs=[pl.BlockSpec((BE, H // 2), lambda i: (i, 0)),
                  pl.BlockSpec((8, H), lambda i: (0, 0)),
                  pl.BlockSpec((8, H), lambda i: (0, 0)),
                  pl.BlockSpec((H, H), lambda i: (0, 0))],
        out_specs=[pl.BlockSpec((BE, H // 2), lambda i: (i, 0)),
                   pl.BlockSpec((8, H), lambda i: (0, 0))],
        out_shape=[jax.ShapeDtypeStruct((EP, H // 2), jnp.uint32),
                   jax.ShapeDtypeStruct((8, H), jnp.float32)],
        scratch_shapes=[pltpu.VMEM((8, H), jnp.float32)],
    )(eh1, stats1, vec, w2)


def _node_xnew(nagg_ref, x_ref, w1_ref, w2_ref, vec_ref):
    h0 = nagg_ref[...] + x_ref[...]
    y = _mmb(h0, w1_ref[...]) + vec_ref[0:1, :]
    m = jnp.mean(y, axis=0, keepdims=True)
    v = jnp.mean(y * y, axis=0, keepdims=True) - m * m
    h = jnp.maximum(vec_ref[1:2, :] * (y - m) * jax.lax.rsqrt(v + EPS)
                    + vec_ref[2:3, :], 0.0)
    y2 = _mmb(h, w2_ref[...]) + vec_ref[3:4, :]
    m2 = jnp.mean(y2, axis=0, keepdims=True)
    v2 = jnp.mean(y2 * y2, axis=0, keepdims=True) - m2 * m2
    return jnp.maximum(vec_ref[4:5, :] * (y2 - m2)
                       * jax.lax.rsqrt(v2 + EPS) + vec_ref[5:6, :], 0.0)


def _node_body(nagg_ref, x_ref, w1_ref, w2_ref, vec_ref, out_ref):
    out_ref[...] = _node_xnew(nagg_ref, x_ref, w1_ref, w2_ref, vec_ref)


def _node(nagg, x, w1, w2, vec):
    return pl.pallas_call(
        _node_body,
        in_specs=[pl.BlockSpec((N, D), lambda: (0, 0)),
                  pl.BlockSpec((N, D), lambda: (0, 0)),
                  pl.BlockSpec((D, H), lambda: (0, 0)),
                  pl.BlockSpec((H, H), lambda: (0, 0)),
                  pl.BlockSpec((8, H), lambda: (0, 0))],
        out_specs=pl.BlockSpec((N, D), lambda: (0, 0)),
        out_shape=jax.ShapeDtypeStruct((N, D), jnp.float32),
    )(nagg, x, w1, w2, vec)


def _node_fused_body(nagg_ref, x_ref, w1_ref, w2_ref, vec_ref, wcat_ref,
                     bcat_ref, out_ref, tabA_ref, tabB_ref):
    xn = _node_xnew(nagg_ref, x_ref, w1_ref, w2_ref, vec_ref)
    out_ref[...] = xn
    t = _mm(xn, wcat_ref[...]) + bcat_ref[0:1, :]
    tabA_ref[...] = t
    tabB_ref[...] = t[:, D:]


def _node_fused(nagg, x, w1, w2, vec, wcat, bcat):
    return pl.pallas_call(
        _node_fused_body,
        in_specs=[pl.BlockSpec((N, D), lambda: (0, 0)),
                  pl.BlockSpec((N, D), lambda: (0, 0)),
                  pl.BlockSpec((D, H), lambda: (0, 0)),
                  pl.BlockSpec((H, H), lambda: (0, 0)),
                  pl.BlockSpec((8, H), lambda: (0, 0)),
                  pl.BlockSpec((D, 2 * D), lambda: (0, 0)),
                  pl.BlockSpec((8, 2 * D), lambda: (0, 0))],
        out_specs=[pl.BlockSpec((N, D), lambda: (0, 0)),
                   pl.BlockSpec((N, 2 * D), lambda: (0, 0)),
                   pl.BlockSpec((N, H), lambda: (0, 0))],
        out_shape=[jax.ShapeDtypeStruct((N, D), jnp.float32),
                   jax.ShapeDtypeStruct((N, 2 * D), jnp.float32),
                   jax.ShapeDtypeStruct((N, H), jnp.float32)],
    )(nagg, x, w1, w2, vec, wcat, bcat)


def _bnrelu_body(eh2_ref, stats_ref, vec_ref, out_ref):
    i = pl.program_id(0)
    rows = i * BE + jax.lax.broadcasted_iota(jnp.int32, (BE, 1), 0)
    v = _bn_from_stats(_unpack2(eh2_ref[...]), stats_ref,
                       vec_ref[0:1, :], vec_ref[1:2, :])
    out_ref[...] = jnp.where(rows < E, v, 0.0)


def _bnrelu(eh2, stats, vec):
    return pl.pallas_call(
        _bnrelu_body,
        grid=(NBLK,),
        in_specs=[pl.BlockSpec((BE, H // 2), lambda i: (i, 0)),
                  pl.BlockSpec((8, H), lambda i: (0, 0)),
                  pl.BlockSpec((8, H), lambda i: (0, 0))],
        out_specs=pl.BlockSpec((BE, H), lambda i: (i, 0)),
        out_shape=jax.ShapeDtypeStruct((EP, H), jnp.float32),
    )(eh2, stats, vec)


def _pool_body(nblk, limit, width, data_ref, gidx_ref, sum_ref, cnt_ref,
               accs, accc):
    i = pl.program_id(0)

    @pl.when(i == 0)
    def _():
        accs[...] = jnp.zeros((B, width), jnp.float32)
        accc[...] = jnp.zeros((B, 128), jnp.float32)

    d = data_ref[...]
    giota = jax.lax.broadcasted_iota(jnp.int32, (B, 1), 0)
    lane = jax.lax.broadcasted_iota(jnp.int32, (1, 128), 1)
    for r in range(BE // 128):
        gr = gidx_ref[r:r + 1, :]
        ids = i * BE + r * 128 + lane
        oh = jnp.where((gr == giota) & (ids < limit), 1.0, 0.0)
        accs[...] = accs[...] + _mm(oh, d[r * 128:(r + 1) * 128, :])
        accc[...] = accc[...] + oh

    @pl.when(i == nblk - 1)
    def _():
        sum_ref[...] = accs[...]
        cnt = jnp.sum(accc[...], axis=1, keepdims=True)
        cnt_ref[...] = jnp.broadcast_to(cnt, (B, 128))


def _pool(limit, data, gidx):
    nblk = data.shape[0] // BE
    width = data.shape[1]
    return pl.pallas_call(
        functools.partial(_pool_body, nblk, limit, width),
        grid=(nblk,),
        in_specs=[pl.BlockSpec((BE, width), lambda i: (i, 0)),
                  pl.BlockSpec((BE // 128, 128), lambda i: (i, 0))],
        out_specs=[pl.BlockSpec((B, width), lambda i: (0, 0)),
                   pl.BlockSpec((B, 128), lambda i: (0, 0))],
        out_shape=[jax.ShapeDtypeStruct((B, width), jnp.float32),
                   jax.ShapeDtypeStruct((B, 128), jnp.float32)],
        scratch_shapes=[pltpu.VMEM((B, width), jnp.float32),
                        pltpu.VMEM((B, 128), jnp.float32)],
    )(data, gidx)


def _final_body(pn_ref, cn_ref, pe_ref, ce_ref, w1n_ref, w1e_ref, w2_ref,
                vb_ref, b2_ref, out_ref):
    pn = pn_ref[...] / jnp.maximum(cn_ref[:, 0:1], 1.0)
    pe = pe_ref[...] / jnp.maximum(ce_ref[:, 0:1], 1.0)
    hn = jnp.maximum(_mm(pn, w1n_ref[...]) + vb_ref[0:1, :], 0.0)
    he = jnp.maximum(_mm(pe, w1e_ref[...]) + vb_ref[1:2, :], 0.0)
    out_ref[...] = (_mm(hn, w2_ref[0:FH, :]) + _mm(he, w2_ref[FH:, :])
                    + b2_ref[0:1, :])


def _final(pn, cn, pe, ce, w1n, w1e, w2, vb, b2):
    return pl.pallas_call(
        _final_body,
        in_specs=[pl.BlockSpec((B, H), lambda: (0, 0)),
                  pl.BlockSpec((B, 128), lambda: (0, 0)),
                  pl.BlockSpec((B, H), lambda: (0, 0)),
                  pl.BlockSpec((B, H), lambda: (0, 0)),
                  pl.BlockSpec((H, FH), lambda: (0, 0)),
                  pl.BlockSpec((H, FH), lambda: (0, 0)),
                  pl.BlockSpec((2 * FH, OUT), lambda: (0, 0)),
                  pl.BlockSpec((8, FH), lambda: (0, 0)),
                  pl.BlockSpec((8, OUT), lambda: (0, 0))],
        out_specs=pl.BlockSpec((B, OUT), lambda: (0, 0)),
        out_shape=jax.ShapeDtypeStruct((B, OUT), jnp.float32),
    )(pn, cn, pe, ce, w1n, w1e, w2, vb, b2)


def _vecs8(*rows, width=H):
    out = jnp.zeros((8, width), jnp.float32)
    for r, v in enumerate(rows):
        out = out.at[r, :].set(v)
    return out


def kernel(x, edge_attr, params, edge_index, batch, num_graphs):
    p = params
    src = edge_index[0]
    dst = edge_index[1]
    src_pad = jnp.pad(src, (0, EP - E))
    dst_pad = jnp.pad(dst, (0, EP - E))
    src2d_g = src_pad.reshape(NCH2, CH2)
    dst2d_g = dst_pad.reshape(NCH2, CH2)
    src2d_s = src_pad.reshape(NCH, 128)
    dst2d_s = dst_pad.reshape(NCH, 128)
    epre = jnp.pad(edge_attr, ((0, EP - E), (0, 0)))
    zrows = jnp.zeros((NROW_CP, 128), jnp.float32)
    zstats = jnp.zeros((8, H), jnp.float32)

    def _wcats(l):
        Wx = p["msg_W"][l][:D]
        xcat_W = jnp.concatenate([Wx, p["e1_W"][l]],
                                 axis=1).astype(jnp.bfloat16)
        bcat = _vecs8(jnp.concatenate([p["msg_b"][l], jnp.zeros((H,))]),
                      width=2 * D)
        return xcat_W, bcat

    xcat_W, bcat = _wcats(0)
    tabA, tabB = _node_pre(x, xcat_W, bcat)

    bn_state = None
    for l in range(L):
        We = p["msg_W"][l][D:]
        gsrc, gdst = _gather(tabA, tabB, src2d_g, dst2d_g)

        wcat = jnp.concatenate([We, p["e1_W"][l]],
                               axis=1).astype(jnp.bfloat16)
        if bn_state is None:
            vec1 = _vecs8(p["e1_b"][l])
            msg, eh1, stats1 = _edge1(False, epre, gsrc, gdst, wcat, vec1,
                                      zstats)
        else:
            pstats, pg, pbe = bn_state
            vec1 = _vecs8(p["e1_b"][l], pg, pbe)
            msg, eh1, stats1 = _edge1(True, epre, gsrc, gdst, wcat, vec1,
                                      pstats)

        nagg = _scatter(msg, dst2d_s, zrows)

        vec2 = _vecs8(p["e1_g"][l], p["e1_be"][l], p["e2_b"][l])
        eh2, stats2 = _edge2(eh1, stats1, vec2,
                             p["e2_W"][l].astype(jnp.bfloat16))

        vecn = _vecs8(p["n1_b"][l], p["n1_g"][l], p["n1_be"][l],
                      p["n2_b"][l], p["n2_g"][l], p["n2_be"][l])
        x = _node(nagg, x, p["n1_W"][l].astype(jnp.bfloat16),
                  p["n2_W"][l].astype(jnp.bfloat16), vecn)
        if l < L - 1:
            xcat_W, bcat = _wcats(l + 1)
            tabA, tabB = _node_pre(x, xcat_W, bcat)

        epre = eh2
        bn_state = (stats2, p["e2_g"][l], p["e2_be"][l])

    x_pad = jnp.pad(x, ((0, 10240 - N), (0, 0)))
    batch2d = jnp.pad(batch, (0, 10240 - N), constant_values=B).reshape(80, 128)
    pn_sum, cn = _pool(N, x_pad, batch2d)

    stats2, pg, pbe = bn_state
    efin = _bnrelu(epre, stats2, _vecs8(pg, pbe))
    sn = _scatter(efin, src2d_s, zrows)
    deg = _scatter_ones(jnp.ones((128, 128), jnp.float32), src2d_s, zrows)
    pe_sum, _ = _pool(N, jnp.pad(sn, ((0, 240, ), (0, 0))), batch2d)
    ce, _ = _pool(N, jnp.pad(deg, ((0, 240), (0, 0))), batch2d)

    vb = _vecs8(p["lin1n_b"], p["lin1e_b"], width=FH)
    b2 = _vecs8(p["lin2_b"], width=OUT)
    out = _final(pn_sum, cn, pe_sum, ce, p["lin1n_W"], p["lin1e_W"],
                 p["lin2_W"], vb, b2)
    return out


# speedup vs baseline: 1.1872x; 1.0266x over previous
import functools

import jax
import jax.numpy as jnp
from jax import lax
from jax.experimental import pallas as pl
from jax.experimental.pallas import tpu as pltpu
from jax.experimental.pallas import tpu_sc as plsc

N = 10000
E = 160000
D = 256
H = 256
L = 4
B = 64
FH = 512
OUT = 128
EPS = 1e-5

EP = 163840
NCH = EP // 128
NW = 32
CHW = NCH // NW
CHT = NCH // 16
BE = 4096
NBLK = EP // BE
NROW_OFF = 624
NROW_CP = 640

_MESH = plsc.VectorSubcoreMesh(core_axis_name="c", subcore_axis_name="s")


def _mm(a, b):
    return jax.lax.dot_general(a, b, (((1,), (0,)), ((), ())),
                               preferred_element_type=jnp.float32)


def _mmb(a, b):
    return jax.lax.dot_general(a.astype(jnp.bfloat16), b,
                               (((1,), (0,)), ((), ())),
                               preferred_element_type=jnp.float32)


CH2 = 128
NCH2 = EP // CH2
CHW_A = 64
CHW_B = NCH2 // 16 - CHW_A


def _gather_body(tabA, tabB, src2d, dst2d, gsrc, gdst, idxs, idxd,
                 bufA, bufB, semG):
    c = lax.axis_index("c")
    s = lax.axis_index("s")
    base = jnp.where(c == 0, s * CHW_A, 16 * CHW_A + s * CHW_B)

    @pl.when(c == 0)
    def _():
        pltpu.sync_copy(src2d.at[pl.ds(s * CHW_A, CHW_A)],
                        idxs.at[pl.ds(0, CHW_A)])
        pltpu.sync_copy(dst2d.at[pl.ds(s * CHW_A, CHW_A)],
                        idxd.at[pl.ds(0, CHW_A)])

    @pl.when(c == 1)
    def _():
        pltpu.sync_copy(src2d.at[pl.ds(16 * CHW_A + s * CHW_B, CHW_B)],
                        idxs.at[pl.ds(0, CHW_B)])
        pltpu.sync_copy(dst2d.at[pl.ds(16 * CHW_A + s * CHW_B, CHW_B)],
                        idxd.at[pl.ds(0, CHW_B)])

    def body(i, _):
        c0 = 2 * i
        c1 = 2 * i + 1
        dA0 = pltpu.async_copy(tabA.at[idxs.at[c0]], bufA.at[0], semG)
        dA1 = pltpu.async_copy(tabA.at[idxs.at[c1]], bufA.at[1], semG)
        dB0 = pltpu.async_copy(tabB.at[idxd.at[c0]], bufB.at[0], semG)
        dB1 = pltpu.async_copy(tabB.at[idxd.at[c1]], bufB.at[1], semG)
        dA0.wait()
        dA1.wait()
        dB0.wait()
        dB1.wait()
        row0 = (base + c0) * CH2
        pltpu.sync_copy(bufA.at[0], gsrc.at[pl.ds(row0, CH2)])
        pltpu.sync_copy(bufA.at[1], gsrc.at[pl.ds(row0 + CH2, CH2)])
        pltpu.sync_copy(bufB.at[0], gdst.at[pl.ds(row0, CH2)])
        pltpu.sync_copy(bufB.at[1], gdst.at[pl.ds(row0 + CH2, CH2)])
        return 0

    npairs = jnp.where(c == 0, CHW_A // 2, CHW_B // 2)
    lax.fori_loop(0, npairs, body, 0)


_gather = pl.kernel(
    _gather_body,
    out_type=[jax.ShapeDtypeStruct((EP, D), jnp.uint32),
              jax.ShapeDtypeStruct((EP, H // 2), jnp.uint32)],
    mesh=_MESH,
    scratch_types=[pltpu.VMEM((CHW_A, CH2), jnp.int32),
                   pltpu.VMEM((CHW_A, CH2), jnp.int32),
                   pltpu.VMEM((2, CH2, D), jnp.uint32),
                   pltpu.VMEM((2, CH2, H // 2), jnp.uint32),
                   pltpu.SemaphoreType.DMA],
)


def _scatter_body(msg, dst2d, zrows, nagg, idxd, buf, accum, semR):
    c = lax.axis_index("c")
    s = lax.axis_index("s")
    coff = c * 128
    pltpu.sync_copy(zrows, accum.at[pl.ds(s * NROW_OFF, NROW_CP)])
    plsc.subcore_barrier()
    pltpu.sync_copy(dst2d.at[pl.ds(s * CHT, CHT)], idxd)

    def body(i, _):
        c0 = 2 * i
        c1 = 2 * i + 1
        row0 = s * (CHT * 128) + c0 * 128
        d0 = pltpu.async_copy(msg.at[pl.ds(row0, 128), pl.ds(coff, 128)],
                              buf.at[0], semR)
        d1 = pltpu.async_copy(msg.at[pl.ds(row0 + 128, 128),
                                     pl.ds(coff, 128)], buf.at[1], semR)
        d0.wait()
        d1.wait()
        pltpu.sync_copy(buf.at[0], accum.at[idxd.at[c0]], add=True)
        pltpu.sync_copy(buf.at[1], accum.at[idxd.at[c1]], add=True)
        return 0

    lax.fori_loop(0, CHT // 2, body, 0)
    plsc.subcore_barrier()
    pltpu.sync_copy(accum.at[pl.ds(s * NROW_OFF, NROW_CP)],
                    nagg.at[pl.ds(s * NROW_OFF, NROW_CP), pl.ds(coff, 128)])


_scatter = pl.kernel(
    _scatter_body,
    out_type=jax.ShapeDtypeStruct((N, D), jnp.float32),
    mesh=_MESH,
    scratch_types=[pltpu.VMEM((CHT, 128), jnp.int32),
                   pltpu.VMEM((2, 128, 128), jnp.float32),
                   pltpu.MemorySpace.VMEM_SHARED((N, 128), jnp.float32),
                   pltpu.SemaphoreType.DMA],
)


def _scatter_ones_body(ones128, dst2d, zrows, nagg, idxd, buf, accum):
    c = lax.axis_index("c")
    s = lax.axis_index("s")
    coff = c * 128
    pltpu.sync_copy(zrows, accum.at[pl.ds(s * NROW_OFF, NROW_CP)])
    plsc.subcore_barrier()
    pltpu.sync_copy(dst2d.at[pl.ds(s * CHT, CHT)], idxd)
    pltpu.sync_copy(ones128, buf)

    def body(ci, _):
        @pl.when(s * CHT + ci < E // 128)
        def _():
            pltpu.sync_copy(buf, accum.at[idxd.at[ci]], add=True)

        return 0

    lax.fori_loop(0, CHT, body, 0)
    plsc.subcore_barrier()
    pltpu.sync_copy(accum.at[pl.ds(s * NROW_OFF, NROW_CP)],
                    nagg.at[pl.ds(s * NROW_OFF, NROW_CP), pl.ds(coff, 128)])


_scatter_ones = pl.kernel(
    _scatter_ones_body,
    out_type=jax.ShapeDtypeStruct((N, D), jnp.float32),
    mesh=_MESH,
    scratch_types=[pltpu.VMEM((CHT, 128), jnp.int32),
                   pltpu.VMEM((128, 128), jnp.float32),
                   pltpu.MemorySpace.VMEM_SHARED((N, 128), jnp.float32)],
)


def _rb(x):
    return jax.lax.bitcast_convert_type(
        x.astype(jnp.bfloat16).astype(jnp.float32), jnp.uint32)


def _pack2(hi, lo):
    return jax.lax.bitwise_or(
        _rb(hi), jax.lax.shift_right_logical(_rb(lo), jnp.uint32(16)))


def _lo_f32(pk):
    return jax.lax.bitcast_convert_type(
        jax.lax.shift_left(pk, jnp.uint32(16)), jnp.float32)


def _hi_f32(pk):
    return jax.lax.bitcast_convert_type(
        jax.lax.bitwise_and(pk, jnp.uint32(0xFFFF0000)), jnp.float32)


def _node_pre_body(x_ref, w_ref, b_ref, tabA_ref, tabB_ref):
    t = _mmb(x_ref[...], w_ref[...]) + b_ref[0:1, :]
    xm = t[:, :D]
    xe = t[:, D:]
    tabA_ref[...] = _pack2(xe, xm)
    tabB_ref[...] = _pack2(xe[:, H // 2:], xe[:, :H // 2])


def _node_pre(x, wcat, bcat):
    return pl.pallas_call(
        _node_pre_body,
        grid=(5,),
        in_specs=[pl.BlockSpec((2000, D), lambda i: (i, 0)),
                  pl.BlockSpec((D, 2 * D), lambda i: (0, 0)),
                  pl.BlockSpec((8, 2 * D), lambda i: (0, 0))],
        out_specs=[pl.BlockSpec((2000, D), lambda i: (i, 0)),
                   pl.BlockSpec((2000, H // 2), lambda i: (i, 0))],
        out_shape=[jax.ShapeDtypeStruct((N, D), jnp.uint32),
                   jax.ShapeDtypeStruct((N, H // 2), jnp.uint32)],
    )(x, wcat, bcat)


def _bn_from_stats(t, stats_ref, g, be):
    m = stats_ref[0:1, :] / E
    v = stats_ref[1:2, :] / E - m * m
    inv = jax.lax.rsqrt(v + EPS)
    return jnp.maximum(g * (t - m) * inv + be, 0.0)


def _unpack2(pk):
    return jnp.concatenate([_lo_f32(pk), _hi_f32(pk)], axis=1)


def _pack_cols(t):
    w = t.shape[1] // 2
    return _pack2(t[:, w:], t[:, :w])


def _edge1_body(apply_bn, epre_ref, gsrc_ref, gdst_ref, wcat_ref, vec_ref,
                pstats_ref, msg_ref, eh1_ref, stats_ref, acc):
    i = pl.program_id(0)

    @pl.when(i == 0)
    def _():
        acc[...] = jnp.zeros((8, H), jnp.float32)

    if apply_bn:
        ep = _unpack2(epre_ref[...])
        ebn = _bn_from_stats(ep, pstats_ref, vec_ref[1:2, :], vec_ref[2:3, :])
    else:
        ebn = epre_ref[...]
    t = _mmb(ebn, wcat_ref[...])
    gp = gsrc_ref[...]
    dp = gdst_ref[...]
    xm = _lo_f32(gp)
    xs = _hi_f32(gp)
    xd = jnp.concatenate([_lo_f32(dp), _hi_f32(dp)], axis=1)
    rows = i * BE + jax.lax.broadcasted_iota(jnp.int32, (BE, 1), 0)
    mask = rows < E
    msg = jnp.maximum(t[:, :D] + xm, 0.0)
    msg_ref[...] = jnp.where(mask, msg, 0.0)
    eh1 = t[:, D:] + xs + xd + vec_ref[0:1, :]
    eh1_ref[...] = _pack_cols(eh1)
    mm_ = jnp.where(mask, eh1, 0.0)
    acc[0:1, :] = acc[0:1, :] + jnp.sum(mm_, axis=0, keepdims=True)
    acc[1:2, :] = acc[1:2, :] + jnp.sum(mm_ * mm_, axis=0, keepdims=True)

    @pl.when(i == NBLK - 1)
    def _():
        stats_ref[...] = acc[...]


def _edge1(apply_bn, epre, gsrc, gdst, wcat, vec, pstats):
    ep_w = H // 2 if apply_bn else D
    return pl.pallas_call(
        functools.partial(_edge1_body, apply_bn),
        grid=(NBLK,),
        in_specs=[pl.BlockSpec((BE, ep_w), lambda i: (i, 0)),
                  pl.BlockSpec((BE, D), lambda i: (i, 0)),
                  pl.BlockSpec((BE, H // 2), lambda i: (i, 0)),
                  pl.BlockSpec((D, 2 * D), lambda i: (0, 0)),
                  pl.BlockSpec((8, H), lambda i: (0, 0)),
                  pl.BlockSpec((8, H), lambda i: (0, 0))],
        out_specs=[pl.BlockSpec((BE, D), lambda i: (i, 0)),
                   pl.BlockSpec((BE, H // 2), lambda i: (i, 0)),
                   pl.BlockSpec((8, H), lambda i: (0, 0))],
        out_shape=[jax.ShapeDtypeStruct((EP, D), jnp.float32),
                   jax.ShapeDtypeStruct((EP, H // 2), jnp.uint32),
                   jax.ShapeDtypeStruct((8, H), jnp.float32)],
        scratch_shapes=[pltpu.VMEM((8, H), jnp.float32)],
    )(epre, gsrc, gdst, wcat, vec, pstats)


def _edge2_body(eh1_ref, stats1_ref, vec_ref, w2_ref, eh2_ref, stats2_ref, acc):
    i = pl.program_id(0)

    @pl.when(i == 0)
    def _():
        acc[...] = jnp.zeros((8, H), jnp.float32)

    a = _bn_from_stats(_unpack2(eh1_ref[...]), stats1_ref,
                       vec_ref[0:1, :], vec_ref[1:2, :])
    t = _mmb(a, w2_ref[...]) + vec_ref[2:3, :]
    eh2_ref[...] = _pack_cols(t)
    rows = i * BE + jax.lax.broadcasted_iota(jnp.int32, (BE, 1), 0)
    mask = rows < E
    mm_ = jnp.where(mask, t, 0.0)
    acc[0:1, :] = acc[0:1, :] + jnp.sum(mm_, axis=0, keepdims=True)
    acc[1:2, :] = acc[1:2, :] + jnp.sum(mm_ * mm_, axis=0, keepdims=True)

    @pl.when(i == NBLK - 1)
    def _():
        stats2_ref[...] = acc[...]


def _edge2(eh1, stats1, vec, w2):
    return pl.pallas_call(
        _edge2_body,
        grid=(NBLK,),
        in_specs=[pl.BlockSpec((BE, H // 2), lambda i: (i, 0)),
                  pl.BlockSpec((8, H), lambda i: (0, 0)),
                  pl.BlockSpec((8, H), lambda i: (0, 0)),
                  pl.BlockSpec((H, H), lambda i: (0, 0))],
        out_specs=[pl.BlockSpec((BE, H // 2), lambda i: (i, 0)),
                   pl.BlockSpec((8, H), lambda i: (0, 0))],
        out_shape=[jax.ShapeDtypeStruct((EP, H // 2), jnp.uint32),
                   jax.ShapeDtypeStruct((8, H), jnp.float32)],
        scratch_shapes=[pltpu.VMEM((8, H), jnp.float32)],
    )(eh1, stats1, vec, w2)


def _node_xnew(nagg_ref, x_ref, w1_ref, w2_ref, vec_ref):
    h0 = nagg_ref[...] + x_ref[...]
    y = _mmb(h0, w1_ref[...]) + vec_ref[0:1, :]
    m = jnp.mean(y, axis=0, keepdims=True)
    v = jnp.mean(y * y, axis=0, keepdims=True) - m * m
    h = jnp.maximum(vec_ref[1:2, :] * (y - m) * jax.lax.rsqrt(v + EPS)
                    + vec_ref[2:3, :], 0.0)
    y2 = _mmb(h, w2_ref[...]) + vec_ref[3:4, :]
    m2 = jnp.mean(y2, axis=0, keepdims=True)
    v2 = jnp.mean(y2 * y2, axis=0, keepdims=True) - m2 * m2
    return jnp.maximum(vec_ref[4:5, :] * (y2 - m2)
                       * jax.lax.rsqrt(v2 + EPS) + vec_ref[5:6, :], 0.0)


def _node_body(nagg_ref, x_ref, w1_ref, w2_ref, vec_ref, out_ref):
    out_ref[...] = _node_xnew(nagg_ref, x_ref, w1_ref, w2_ref, vec_ref)


def _node(nagg, x, w1, w2, vec):
    return pl.pallas_call(
        _node_body,
        in_specs=[pl.BlockSpec((N, D), lambda: (0, 0)),
                  pl.BlockSpec((N, D), lambda: (0, 0)),
                  pl.BlockSpec((D, H), lambda: (0, 0)),
                  pl.BlockSpec((H, H), lambda: (0, 0)),
                  pl.BlockSpec((8, H), lambda: (0, 0))],
        out_specs=pl.BlockSpec((N, D), lambda: (0, 0)),
        out_shape=jax.ShapeDtypeStruct((N, D), jnp.float32),
    )(nagg, x, w1, w2, vec)


def _node_fused_body(nagg_ref, x_ref, w1_ref, w2_ref, vec_ref, wcat_ref,
                     bcat_ref, out_ref, tabA_ref, tabB_ref):
    xn = _node_xnew(nagg_ref, x_ref, w1_ref, w2_ref, vec_ref)
    out_ref[...] = xn
    t = _mm(xn, wcat_ref[...]) + bcat_ref[0:1, :]
    tabA_ref[...] = t
    tabB_ref[...] = t[:, D:]


def _node_fused(nagg, x, w1, w2, vec, wcat, bcat):
    return pl.pallas_call(
        _node_fused_body,
        in_specs=[pl.BlockSpec((N, D), lambda: (0, 0)),
                  pl.BlockSpec((N, D), lambda: (0, 0)),
                  pl.BlockSpec((D, H), lambda: (0, 0)),
                  pl.BlockSpec((H, H), lambda: (0, 0)),
                  pl.BlockSpec((8, H), lambda: (0, 0)),
                  pl.BlockSpec((D, 2 * D), lambda: (0, 0)),
                  pl.BlockSpec((8, 2 * D), lambda: (0, 0))],
        out_specs=[pl.BlockSpec((N, D), lambda: (0, 0)),
                   pl.BlockSpec((N, 2 * D), lambda: (0, 0)),
                   pl.BlockSpec((N, H), lambda: (0, 0))],
        out_shape=[jax.ShapeDtypeStruct((N, D), jnp.float32),
                   jax.ShapeDtypeStruct((N, 2 * D), jnp.float32),
                   jax.ShapeDtypeStruct((N, H), jnp.float32)],
    )(nagg, x, w1, w2, vec, wcat, bcat)


def _bnrelu_body(eh2_ref, stats_ref, vec_ref, out_ref):
    i = pl.program_id(0)
    rows = i * BE + jax.lax.broadcasted_iota(jnp.int32, (BE, 1), 0)
    v = _bn_from_stats(_unpack2(eh2_ref[...]), stats_ref,
                       vec_ref[0:1, :], vec_ref[1:2, :])
    out_ref[...] = jnp.where(rows < E, v, 0.0)


def _bnrelu(eh2, stats, vec):
    return pl.pallas_call(
        _bnrelu_body,
        grid=(NBLK,),
        in_specs=[pl.BlockSpec((BE, H // 2), lambda i: (i, 0)),
                  pl.BlockSpec((8, H), lambda i: (0, 0)),
                  pl.BlockSpec((8, H), lambda i: (0, 0))],
        out_specs=pl.BlockSpec((BE, H), lambda i: (i, 0)),
        out_shape=jax.ShapeDtypeStruct((EP, H), jnp.float32),
    )(eh2, stats, vec)


def _pool_body(nblk, limit, width, data_ref, gidx_ref, sum_ref, cnt_ref,
               accs, accc):
    i = pl.program_id(0)

    @pl.when(i == 0)
    def _():
        accs[...] = jnp.zeros((B, width), jnp.float32)
        accc[...] = jnp.zeros((B, 128), jnp.float32)

    d = data_ref[...]
    giota = jax.lax.broadcasted_iota(jnp.int32, (B, 1), 0)
    lane = jax.lax.broadcasted_iota(jnp.int32, (1, 128), 1)
    for r in range(BE // 128):
        gr = gidx_ref[r:r + 1, :]
        ids = i * BE + r * 128 + lane
        oh = jnp.where((gr == giota) & (ids < limit), 1.0, 0.0)
        accs[...] = accs[...] + _mm(oh, d[r * 128:(r + 1) * 128, :])
        accc[...] = accc[...] + oh

    @pl.when(i == nblk - 1)
    def _():
        sum_ref[...] = accs[...]
        cnt = jnp.sum(accc[...], axis=1, keepdims=True)
        cnt_ref[...] = jnp.broadcast_to(cnt, (B, 128))


def _pool(limit, data, gidx):
    nblk = data.shape[0] // BE
    width = data.shape[1]
    return pl.pallas_call(
        functools.partial(_pool_body, nblk, limit, width),
        grid=(nblk,),
        in_specs=[pl.BlockSpec((BE, width), lambda i: (i, 0)),
                  pl.BlockSpec((BE // 128, 128), lambda i: (i, 0))],
        out_specs=[pl.BlockSpec((B, width), lambda i: (0, 0)),
                   pl.BlockSpec((B, 128), lambda i: (0, 0))],
        out_shape=[jax.ShapeDtypeStruct((B, width), jnp.float32),
                   jax.ShapeDtypeStruct((B, 128), jnp.float32)],
        scratch_shapes=[pltpu.VMEM((B, width), jnp.float32),
                        pltpu.VMEM((B, 128), jnp.float32)],
    )(data, gidx)


def _final_body(pn_ref, cn_ref, pe_ref, ce_ref, w1n_ref, w1e_ref, w2_ref,
                vb_ref, b2_ref, out_ref):
    pn = pn_ref[...] / jnp.maximum(cn_ref[:, 0:1], 1.0)
    pe = pe_ref[...] / jnp.maximum(ce_ref[:, 0:1], 1.0)
    hn = jnp.maximum(_mm(pn, w1n_ref[...]) + vb_ref[0:1, :], 0.0)
    he = jnp.maximum(_mm(pe, w1e_ref[...]) + vb_ref[1:2, :], 0.0)
    out_ref[...] = (_mm(hn, w2_ref[0:FH, :]) + _mm(he, w2_ref[FH:, :])
                    + b2_ref[0:1, :])


def _final(pn, cn, pe, ce, w1n, w1e, w2, vb, b2):
    return pl.pallas_call(
        _final_body,
        in_specs=[pl.BlockSpec((B, H), lambda: (0, 0)),
                  pl.BlockSpec((B, 128), lambda: (0, 0)),
                  pl.BlockSpec((B, H), lambda: (0, 0)),
                  pl.BlockSpec((B, H), lambda: (0, 0)),
                  pl.BlockSpec((H, FH), lambda: (0, 0)),
                  pl.BlockSpec((H, FH), lambda: (0, 0)),
                  pl.BlockSpec((2 * FH, OUT), lambda: (0, 0)),
                  pl.BlockSpec((8, FH), lambda: (0, 0)),
                  pl.BlockSpec((8, OUT), lambda: (0, 0))],
        out_specs=pl.BlockSpec((B, OUT), lambda: (0, 0)),
        out_shape=jax.ShapeDtypeStruct((B, OUT), jnp.float32),
    )(pn, cn, pe, ce, w1n, w1e, w2, vb, b2)


def _vecs8(*rows, width=H):
    out = jnp.zeros((8, width), jnp.float32)
    for r, v in enumerate(rows):
        out = out.at[r, :].set(v)
    return out


def kernel(x, edge_attr, params, edge_index, batch, num_graphs):
    p = params
    src = edge_index[0]
    dst = edge_index[1]
    src_pad = jnp.pad(src, (0, EP - E))
    dst_pad = jnp.pad(dst, (0, EP - E))
    src2d_g = src_pad.reshape(NCH2, CH2)
    dst2d_g = dst_pad.reshape(NCH2, CH2)
    src2d_s = src_pad.reshape(NCH, 128)
    dst2d_s = dst_pad.reshape(NCH, 128)
    epre = jnp.pad(edge_attr, ((0, EP - E), (0, 0)))
    zrows = jnp.zeros((NROW_CP, 128), jnp.float32)
    zstats = jnp.zeros((8, H), jnp.float32)

    def _wcats(l):
        Wx = p["msg_W"][l][:D]
        xcat_W = jnp.concatenate([Wx, p["e1_W"][l]],
                                 axis=1).astype(jnp.bfloat16)
        bcat = _vecs8(jnp.concatenate([p["msg_b"][l], jnp.zeros((H,))]),
                      width=2 * D)
        return xcat_W, bcat

    xcat_W, bcat = _wcats(0)
    tabA, tabB = _node_pre(x, xcat_W, bcat)

    bn_state = None
    for l in range(L):
        We = p["msg_W"][l][D:]
        gsrc, gdst = _gather(tabA, tabB, src2d_g, dst2d_g)

        wcat = jnp.concatenate([We, p["e1_W"][l]],
                               axis=1).astype(jnp.bfloat16)
        if bn_state is None:
            vec1 = _vecs8(p["e1_b"][l])
            msg, eh1, stats1 = _edge1(False, epre, gsrc, gdst, wcat, vec1,
                                      zstats)
        else:
            pstats, pg, pbe = bn_state
            vec1 = _vecs8(p["e1_b"][l], pg, pbe)
            msg, eh1, stats1 = _edge1(True, epre, gsrc, gdst, wcat, vec1,
                                      pstats)

        nagg = _scatter(msg, dst2d_s, zrows)

        vec2 = _vecs8(p["e1_g"][l], p["e1_be"][l], p["e2_b"][l])
        eh2, stats2 = _edge2(eh1, stats1, vec2,
                             p["e2_W"][l].astype(jnp.bfloat16))

        vecn = _vecs8(p["n1_b"][l], p["n1_g"][l], p["n1_be"][l],
                      p["n2_b"][l], p["n2_g"][l], p["n2_be"][l])
        x = _node(nagg, x, p["n1_W"][l].astype(jnp.bfloat16),
                  p["n2_W"][l].astype(jnp.bfloat16), vecn)
        if l < L - 1:
            xcat_W, bcat = _wcats(l + 1)
            tabA, tabB = _node_pre(x, xcat_W, bcat)

        epre = eh2
        bn_state = (stats2, p["e2_g"][l], p["e2_be"][l])

    x_pad = jnp.pad(x, ((0, 10240 - N), (0, 0)))
    batch2d = jnp.pad(batch, (0, 10240 - N), constant_values=B).reshape(80, 128)
    pn_sum, cn = _pool(N, x_pad, batch2d)

    stats2, pg, pbe = bn_state
    efin = _bnrelu(epre, stats2, _vecs8(pg, pbe))
    sn = _scatter(efin, src2d_s, zrows)
    deg = _scatter_ones(jnp.ones((128, 128), jnp.float32), src2d_s, zrows)
    pe_sum, _ = _pool(N, jnp.pad(sn, ((0, 240, ), (0, 0))), batch2d)
    ce, _ = _pool(N, jnp.pad(deg, ((0, 240), (0, 0))), batch2d)

    vb = _vecs8(p["lin1n_b"], p["lin1e_b"], width=FH)
    b2 = _vecs8(p["lin2_b"], width=OUT)
    out = _final(pn_sum, cn, pe_sum, ce, p["lin1n_W"], p["lin1e_W"],
                 p["lin2_W"], vb, b2)
    return out
